# baseline (speedup 1.0000x reference)
import functools

import jax
import jax.numpy as jnp
from jax import lax
from jax.experimental import pallas as pl
from jax.experimental.pallas import tpu as pltpu
from jax.experimental.pallas import tpu_sc as plsc

N_NODES = 10000
D = 128
N_EDGES = 320000
NC = 2
NS = 16
NW = NC * NS
CH = 128
NCHUNK = 80
EPW = NCHUNK * CH
E_PAD = NW * EPW
NACC = 10240
RPT = NACC // NS
TEMP = 10.0
WGT = 1.0

_mesh = plsc.VectorSubcoreMesh(core_axis_name="c", subcore_axis_name="s")
_f32 = jnp.float32


@functools.partial(
    pl.kernel,
    out_type=[jax.ShapeDtypeStruct((NC * NACC, D), _f32)],
    mesh=_mesh,
    scratch_types=[
        pltpu.VMEM((NCHUNK, CH), jnp.int32),
        pltpu.VMEM((NCHUNK, CH), jnp.int32),
        pltpu.VMEM((CH, D), _f32),
        pltpu.VMEM_SHARED((NACC, D), _f32),
        pltpu.SemaphoreType.DMA,
    ],
)
def _sc_rows(table_h, gidx_h, sidx_h, z2_h, out_h,
             gidx_v, sidx_v, rows_v, acc_r, sem):
    cid = lax.axis_index("c")
    sid = lax.axis_index("s")
    wid = sid * NC + cid
    tb = sid * RPT
    pltpu.sync_copy(z2_h.at[pl.ds(tb, RPT)], acc_r.at[pl.ds(tb, RPT)])
    pltpu.sync_copy(gidx_h.at[wid], gidx_v)
    pltpu.sync_copy(sidx_h.at[wid], sidx_v)
    plsc.subcore_barrier()

    def step(ci, carry):
        pltpu.async_copy(table_h.at[gidx_v.at[ci]], rows_v, sem).wait()
        pltpu.sync_copy(rows_v, acc_r.at[sidx_v.at[ci]], add=True)
        return carry

    lax.fori_loop(0, NCHUNK, step, 0)
    plsc.subcore_barrier()
    pltpu.sync_copy(acc_r.at[pl.ds(tb, RPT)],
                    out_h.at[pl.ds(cid * NACC + tb, RPT)])


@functools.partial(
    pl.kernel,
    out_type=[jax.ShapeDtypeStruct((NC * NACC, D), _f32),
              jax.ShapeDtypeStruct((NC * NACC,), _f32)],
    mesh=_mesh,
    scratch_types=[
        pltpu.VMEM((NCHUNK, CH), jnp.int32),
        pltpu.VMEM((NCHUNK, CH), jnp.int32),
        pltpu.VMEM((CH, D), _f32),
        pltpu.VMEM((CH,), _f32),
        pltpu.VMEM((RPT,), _f32),
        pltpu.VMEM_SHARED((NACC, D), _f32),
        pltpu.VMEM_SHARED((NACC,), _f32),
        pltpu.SemaphoreType.DMA,
    ],
)
def _sc_rows_scal(table_h, tab1_h, gidx_h, sidx_h, z2_h, out_h, out1_h,
                  gidx_v, sidx_v, rows_v, scal_v, bnc_v, acc_r, acc_s, sem):
    cid = lax.axis_index("c")
    sid = lax.axis_index("s")
    wid = sid * NC + cid
    tb = sid * RPT
    pltpu.sync_copy(z2_h.at[pl.ds(tb, RPT)], acc_r.at[pl.ds(tb, RPT)])
    for k in range(RPT // 16):
        bnc_v[pl.ds(k * 16, 16)] = jnp.zeros((16,), _f32)
    pltpu.sync_copy(bnc_v, acc_s.at[pl.ds(tb, RPT)])
    pltpu.sync_copy(gidx_h.at[wid], gidx_v)
    pltpu.sync_copy(sidx_h.at[wid], sidx_v)
    plsc.subcore_barrier()

    def step(ci, carry):
        pltpu.async_copy(table_h.at[gidx_v.at[ci]], rows_v, sem).wait()
        pltpu.sync_copy(rows_v, acc_r.at[sidx_v.at[ci]], add=True)
        pltpu.async_copy(tab1_h.at[gidx_v.at[ci]], scal_v, sem).wait()
        pltpu.sync_copy(scal_v, acc_s.at[sidx_v.at[ci]], add=True)
        return carry

    lax.fori_loop(0, NCHUNK, step, 0)
    plsc.subcore_barrier()
    pltpu.sync_copy(acc_r.at[pl.ds(tb, RPT)],
                    out_h.at[pl.ds(cid * NACC + tb, RPT)])
    pltpu.sync_copy(acc_s.at[pl.ds(tb, RPT)], bnc_v)
    pltpu.sync_copy(bnc_v, out1_h.at[pl.ds(cid * NACC + tb, RPT)])


@functools.partial(
    pl.kernel,
    out_type=[jax.ShapeDtypeStruct((NC * NACC,), _f32),
              jax.ShapeDtypeStruct((NC * NACC,), _f32)],
    mesh=_mesh,
    scratch_types=[
        pltpu.VMEM((NCHUNK, CH), jnp.int32),
        pltpu.VMEM((NCHUNK, CH), jnp.int32),
        pltpu.VMEM((CH,), _f32),
        pltpu.VMEM((RPT,), _f32),
        pltpu.VMEM_SHARED((NACC,), _f32),
        pltpu.VMEM_SHARED((NACC,), _f32),
    ],
)
def _sc_deg(didx_h, oidx_h, outd_h, outo_h,
            didx_v, oidx_v, ones_v, bnc_v, acc_d, acc_o):
    cid = lax.axis_index("c")
    sid = lax.axis_index("s")
    wid = sid * NC + cid
    tb = sid * RPT
    for k in range(RPT // 16):
        bnc_v[pl.ds(k * 16, 16)] = jnp.zeros((16,), _f32)
    pltpu.sync_copy(bnc_v, acc_d.at[pl.ds(tb, RPT)])
    pltpu.sync_copy(bnc_v, acc_o.at[pl.ds(tb, RPT)])
    pltpu.sync_copy(didx_h.at[wid], didx_v)
    pltpu.sync_copy(oidx_h.at[wid], oidx_v)
    for k in range(CH // 16):
        ones_v[pl.ds(k * 16, 16)] = jnp.full((16,), 1.0, _f32)
    plsc.subcore_barrier()

    def step(ci, carry):
        pltpu.sync_copy(ones_v, acc_d.at[didx_v.at[ci]], add=True)
        pltpu.sync_copy(ones_v, acc_o.at[oidx_v.at[ci]], add=True)
        return carry

    lax.fori_loop(0, NCHUNK, step, 0)
    plsc.subcore_barrier()
    pltpu.sync_copy(acc_d.at[pl.ds(tb, RPT)], bnc_v)
    pltpu.sync_copy(bnc_v, outd_h.at[pl.ds(cid * NACC + tb, RPT)])
    pltpu.sync_copy(acc_o.at[pl.ds(tb, RPT)], bnc_v)
    pltpu.sync_copy(bnc_v, outo_h.at[pl.ds(cid * NACC + tb, RPT)])


def _tc(body, n_out):
    shapes = []
    for s in n_out:
        shapes.append(jax.ShapeDtypeStruct(s, _f32))
    return pl.pallas_call(body, out_shape=shapes)


def _deg_body(dp_ref, op_ref, dinv_o, dout_o):
    deg = dp_ref[0] + dp_ref[1] + 1.0
    dinv_o[...] = lax.rsqrt(jnp.maximum(deg, 1e-12))
    dout_o[...] = op_ref[0] + op_ref[1]


def _mm_body(x_ref, w_ref, b_ref, dinv_ref, hh_o, sp_o):
    h = jnp.dot(x_ref[...], w_ref[...], preferred_element_type=_f32)
    dv = dinv_ref[...][:, None]
    hh_o[...] = dv * h
    sp_o[...] = b_ref[...][None, :] + dv * dv * h


def _h1_body(sp_ref, dinv_ref, rp_ref, h1_o, n_o):
    h1 = sp_ref[...] + dinv_ref[...][:, None] * (rp_ref[0] + rp_ref[1])
    h1_o[...] = h1
    n_o[...] = jnp.sum(h1 * h1, axis=1)


def _fin_body(sp_ref, dinv_ref, rp_ref, o_ref):
    o_ref[...] = sp_ref[...] + dinv_ref[...][:, None] * (rp_ref[0] + rp_ref[1])


def _coef_body(h1_ref, n_ref, dout_ref, s1p_ref, t1p_ref, a_o, ah_o, s1_o):
    h1 = h1_ref[...]
    s1 = s1p_ref[0] + s1p_ref[1]
    s1_o[...] = s1
    t1 = t1p_ref[0] + t1p_ref[1]
    e = 0.5 * dout_ref[...] * n_ref[...] - jnp.sum(h1 * s1, axis=1) + 0.5 * t1
    valid = lax.broadcasted_iota(jnp.int32, (NACC,), 0) < N_NODES
    m = jnp.max(jnp.where(valid, e, -jnp.inf))
    c = jnp.maximum(m, 1e-9)
    et = e / c
    z = jnp.where(valid, -et / TEMP, -jnp.inf)
    p = jnp.exp(z - jnp.max(z))
    p = p / jnp.sum(p)
    g = -(jnp.log(p + 1e-12) + p / (p + 1e-12))
    dz = p * (g - jnp.sum(g * p))
    q = -dz / TEMP
    mask = jnp.where(valid & (e == m), 1.0, 0.0)
    nt = jnp.maximum(jnp.sum(mask), 1.0)
    dcdm = jnp.where(m > 1e-9, 1.0, jnp.where(m == 1e-9, 0.5, 0.0))
    beta = dcdm * (-jnp.sum(q * et) / c)
    a = jnp.where(valid, q / c + beta * mask / nt, 0.0)
    a_o[...] = a
    ah_o[...] = a[:, None] * h1


def _grad_body(h1_ref, a_ref, dout_ref, s1_ref, s2p_ref, s3p_ref, o_ref):
    h1 = h1_ref[...]
    s2 = s2p_ref[0] + s2p_ref[1]
    s3 = (s3p_ref[0] + s3p_ref[1])[:, None]
    av = a_ref[...][:, None]
    grd = av * (dout_ref[...][:, None] * h1 - s1_ref[...]) - s2 + s3 * h1
    o_ref[...] = jnp.maximum(h1 + WGT * grd, 0.0)


def kernel(x, edge_index, W1, b1, W2, b2, W_out, b_out):
    src = edge_index[0]
    dst = edge_index[1]
    npad = E_PAD - N_EDGES
    ar = jnp.arange(npad, dtype=jnp.int32)
    pad_g = ar % N_NODES
    pad_s = N_NODES + ar % (NACC - N_NODES)
    src_g = jnp.concatenate([src, pad_g]).reshape(NW, NCHUNK, CH)
    src_s = jnp.concatenate([src, pad_s]).reshape(NW, NCHUNK, CH)
    dst_g = jnp.concatenate([dst, pad_g]).reshape(NW, NCHUNK, CH)
    dst_s = jnp.concatenate([dst, pad_s]).reshape(NW, NCHUNK, CH)
    z2 = jnp.zeros((NACC, D), _f32)
    xp = jnp.pad(x, ((0, NACC - N_NODES), (0, 0)))

    degp, degop = _sc_deg(dst_s, src_s)
    dinv, deg_out = _tc(_deg_body, [(NACC,), (NACC,)])(
        degp.reshape(NC, NACC), degop.reshape(NC, NACC))

    h = xp
    for wgt, bias in ((W1, b1), (W2, b2)):
        hh, sp = _tc(_mm_body, [(NACC, D), (NACC, D)])(h, wgt, bias, dinv)
        rows_a = _sc_rows(hh, src_g, dst_s, z2)[0].reshape(NC, NACC, D)
        h1, n = _tc(_h1_body, [(NACC, D), (NACC,)])(sp, dinv, rows_a)
        rows_b, t1p = _sc_rows_scal(h1, n, dst_g, src_s, z2)
        a, ah, s1 = _tc(_coef_body, [(NACC,), (NACC, D), (NACC, D)])(
            h1, n, deg_out, rows_b.reshape(NC, NACC, D), t1p.reshape(NC, NACC))
        rows_c, s3p = _sc_rows_scal(ah, a, src_g, dst_s, z2)
        h = _tc(_grad_body, [(NACC, D)])(
            h1, a, deg_out, s1, rows_c.reshape(NC, NACC, D),
            s3p.reshape(NC, NACC))[0]

    hh, sp = _tc(_mm_body, [(NACC, D), (NACC, D)])(h, W_out, b_out, dinv)
    rows_f = _sc_rows(hh, src_g, dst_s, z2)[0].reshape(NC, NACC, D)
    emb = _tc(_fin_body, [(NACC, D)])(sp, dinv, rows_f)[0]
    return emb[:N_NODES]


# speedup vs baseline: 1.5481x; 1.5481x over previous
import functools

import jax
import jax.numpy as jnp
from jax import lax
from jax.experimental import pallas as pl
from jax.experimental.pallas import tpu as pltpu
from jax.experimental.pallas import tpu_sc as plsc

N_NODES = 10000
D = 128
N_EDGES = 320000
NC = 2
NS = 16
NW = NC * NS
CH = 128
NCHUNK = 80
EPW = NCHUNK * CH
E_PAD = NW * EPW
NACC = 10240
RPT = NACC // NS
TEMP = 10.0
WGT = 1.0

_mesh = plsc.VectorSubcoreMesh(core_axis_name="c", subcore_axis_name="s")
_f32 = jnp.float32


@functools.partial(
    pl.kernel,
    out_type=[jax.ShapeDtypeStruct((NC * NACC, D), _f32)],
    mesh=_mesh,
    scratch_types=[
        pltpu.VMEM((NCHUNK, CH), jnp.int32),
        pltpu.VMEM((2, 1, CH), jnp.int32),
        pltpu.VMEM((2, CH, D), _f32),
        pltpu.VMEM_SHARED((NACC, D), _f32),
        pltpu.SemaphoreType.DMA,
        pltpu.SemaphoreType.DMA,
        pltpu.SemaphoreType.DMA,
    ],
)
def _sc_rows(table_h, gidx_h, sidx_h, z2_h, out_h,
             gidx_v, sidx_v, rows_v, acc_r, sem_g, sem_i0, sem_i1):
    cid = lax.axis_index("c")
    sid = lax.axis_index("s")
    wid = sid * NC + cid
    tb = sid * RPT
    pltpu.sync_copy(z2_h.at[pl.ds(tb, RPT)], acc_r.at[pl.ds(tb, RPT)])
    pltpu.sync_copy(gidx_h.at[wid], gidx_v)
    plsc.subcore_barrier()
    pltpu.async_copy(table_h.at[gidx_v.at[0]], rows_v.at[0], sem_g)
    pltpu.async_copy(sidx_h.at[wid, pl.ds(0, 1)], sidx_v.at[0], sem_i0)
    pltpu.async_copy(sidx_h.at[wid, pl.ds(1, 1)], sidx_v.at[1], sem_i1)
    isems = (sem_i0, sem_i1)

    def step(g, carry):
        for b in range(2):
            ci = 2 * g + b
            pltpu.make_async_copy(
                table_h.at[gidx_v.at[ci]], rows_v.at[b], sem_g).wait()
            nx = jnp.minimum(ci + 1, NCHUNK - 1)
            pltpu.async_copy(table_h.at[gidx_v.at[nx]], rows_v.at[1 - b], sem_g)
            pltpu.make_async_copy(
                sidx_h.at[wid, pl.ds(ci, 1)], sidx_v.at[b], isems[b]).wait()
            pltpu.sync_copy(rows_v.at[b], acc_r.at[sidx_v.at[b, 0]], add=True)
            nx2 = jnp.minimum(ci + 2, NCHUNK - 1)
            pltpu.async_copy(sidx_h.at[wid, pl.ds(nx2, 1)], sidx_v.at[b],
                             isems[b])
        return carry

    lax.fori_loop(0, NCHUNK // 2, step, 0)
    pltpu.make_async_copy(
        table_h.at[gidx_v.at[NCHUNK - 1]], rows_v.at[0], sem_g).wait()
    pltpu.make_async_copy(
        sidx_h.at[wid, pl.ds(0, 1)], sidx_v.at[0], sem_i0).wait()
    pltpu.make_async_copy(
        sidx_h.at[wid, pl.ds(0, 1)], sidx_v.at[1], sem_i1).wait()
    plsc.subcore_barrier()
    pltpu.sync_copy(acc_r.at[pl.ds(tb, RPT)],
                    out_h.at[pl.ds(cid * NACC + tb, RPT)])


@functools.partial(
    pl.kernel,
    out_type=[jax.ShapeDtypeStruct((NC * NACC, D), _f32),
              jax.ShapeDtypeStruct((NC * NACC,), _f32)],
    mesh=_mesh,
    scratch_types=[
        pltpu.VMEM((NCHUNK, CH), jnp.int32),
        pltpu.VMEM((2, 1, CH), jnp.int32),
        pltpu.VMEM((2, CH, D), _f32),
        pltpu.VMEM((2, CH), _f32),
        pltpu.VMEM((RPT,), _f32),
        pltpu.VMEM_SHARED((NACC, D), _f32),
        pltpu.VMEM_SHARED((NACC,), _f32),
        pltpu.SemaphoreType.DMA,
        pltpu.SemaphoreType.DMA,
        pltpu.SemaphoreType.DMA,
        pltpu.SemaphoreType.DMA,
    ],
)
def _sc_rows_scal(table_h, tab1_h, gidx_h, sidx_h, z2_h, out_h, out1_h,
                  gidx_v, sidx_v, rows_v, scal_v, bnc_v, acc_r, acc_s,
                  sem_r, sem_s, sem_i0, sem_i1):
    cid = lax.axis_index("c")
    sid = lax.axis_index("s")
    wid = sid * NC + cid
    tb = sid * RPT
    pltpu.sync_copy(z2_h.at[pl.ds(tb, RPT)], acc_r.at[pl.ds(tb, RPT)])
    for k in range(RPT // 16):
        bnc_v[pl.ds(k * 16, 16)] = jnp.zeros((16,), _f32)
    pltpu.sync_copy(bnc_v, acc_s.at[pl.ds(tb, RPT)])
    pltpu.sync_copy(gidx_h.at[wid], gidx_v)
    plsc.subcore_barrier()
    pltpu.async_copy(table_h.at[gidx_v.at[0]], rows_v.at[0], sem_r)
    pltpu.async_copy(tab1_h.at[gidx_v.at[0]], scal_v.at[0], sem_s)
    pltpu.async_copy(sidx_h.at[wid, pl.ds(0, 1)], sidx_v.at[0], sem_i0)
    pltpu.async_copy(sidx_h.at[wid, pl.ds(1, 1)], sidx_v.at[1], sem_i1)
    isems = (sem_i0, sem_i1)

    def step(g, carry):
        for b in range(2):
            ci = 2 * g + b
            pltpu.make_async_copy(
                table_h.at[gidx_v.at[ci]], rows_v.at[b], sem_r).wait()
            pltpu.make_async_copy(
                tab1_h.at[gidx_v.at[ci]], scal_v.at[b], sem_s).wait()
            nx = jnp.minimum(ci + 1, NCHUNK - 1)
            pltpu.async_copy(table_h.at[gidx_v.at[nx]], rows_v.at[1 - b], sem_r)
            pltpu.async_copy(tab1_h.at[gidx_v.at[nx]], scal_v.at[1 - b], sem_s)
            pltpu.make_async_copy(
                sidx_h.at[wid, pl.ds(ci, 1)], sidx_v.at[b], isems[b]).wait()
            pltpu.sync_copy(rows_v.at[b], acc_r.at[sidx_v.at[b, 0]], add=True)
            pltpu.sync_copy(scal_v.at[b], acc_s.at[sidx_v.at[b, 0]], add=True)
            nx2 = jnp.minimum(ci + 2, NCHUNK - 1)
            pltpu.async_copy(sidx_h.at[wid, pl.ds(nx2, 1)], sidx_v.at[b],
                             isems[b])
        return carry

    lax.fori_loop(0, NCHUNK // 2, step, 0)
    pltpu.make_async_copy(
        table_h.at[gidx_v.at[NCHUNK - 1]], rows_v.at[0], sem_r).wait()
    pltpu.make_async_copy(
        tab1_h.at[gidx_v.at[NCHUNK - 1]], scal_v.at[0], sem_s).wait()
    pltpu.make_async_copy(
        sidx_h.at[wid, pl.ds(0, 1)], sidx_v.at[0], sem_i0).wait()
    pltpu.make_async_copy(
        sidx_h.at[wid, pl.ds(0, 1)], sidx_v.at[1], sem_i1).wait()
    plsc.subcore_barrier()
    pltpu.sync_copy(acc_r.at[pl.ds(tb, RPT)],
                    out_h.at[pl.ds(cid * NACC + tb, RPT)])
    pltpu.sync_copy(acc_s.at[pl.ds(tb, RPT)], bnc_v)
    pltpu.sync_copy(bnc_v, out1_h.at[pl.ds(cid * NACC + tb, RPT)])


@functools.partial(
    pl.kernel,
    out_type=[jax.ShapeDtypeStruct((NC * NACC,), _f32),
              jax.ShapeDtypeStruct((NC * NACC,), _f32)],
    mesh=_mesh,
    scratch_types=[
        pltpu.VMEM((NCHUNK, CH), jnp.int32),
        pltpu.VMEM((NCHUNK, CH), jnp.int32),
        pltpu.VMEM((CH,), _f32),
        pltpu.VMEM((RPT,), _f32),
        pltpu.VMEM_SHARED((NACC,), _f32),
        pltpu.VMEM_SHARED((NACC,), _f32),
        pltpu.SemaphoreType.DMA,
        pltpu.SemaphoreType.DMA,
    ],
)
def _sc_deg(didx_h, oidx_h, outd_h, outo_h,
            didx_v, oidx_v, ones_v, bnc_v, acc_d, acc_o, sem_d, sem_o):
    cid = lax.axis_index("c")
    sid = lax.axis_index("s")
    wid = sid * NC + cid
    tb = sid * RPT
    for k in range(RPT // 16):
        bnc_v[pl.ds(k * 16, 16)] = jnp.zeros((16,), _f32)
    pltpu.sync_copy(bnc_v, acc_d.at[pl.ds(tb, RPT)])
    pltpu.sync_copy(bnc_v, acc_o.at[pl.ds(tb, RPT)])
    pltpu.sync_copy(didx_h.at[wid], didx_v)
    pltpu.sync_copy(oidx_h.at[wid], oidx_v)
    for k in range(CH // 16):
        ones_v[pl.ds(k * 16, 16)] = jnp.full((16,), 1.0, _f32)
    plsc.subcore_barrier()

    def step(ci, carry):
        pltpu.async_copy(ones_v, acc_d.at[didx_v.at[ci]], sem_d, add=True)
        pltpu.async_copy(ones_v, acc_o.at[oidx_v.at[ci]], sem_o, add=True)
        return carry

    lax.fori_loop(0, NCHUNK, step, 0)

    def drain(ci, carry):
        pltpu.make_async_copy(ones_v, acc_d.at[didx_v.at[ci]], sem_d).wait()
        pltpu.make_async_copy(ones_v, acc_o.at[oidx_v.at[ci]], sem_o).wait()
        return carry

    lax.fori_loop(0, NCHUNK, drain, 0)
    plsc.subcore_barrier()
    pltpu.sync_copy(acc_d.at[pl.ds(tb, RPT)], bnc_v)
    pltpu.sync_copy(bnc_v, outd_h.at[pl.ds(cid * NACC + tb, RPT)])
    pltpu.sync_copy(acc_o.at[pl.ds(tb, RPT)], bnc_v)
    pltpu.sync_copy(bnc_v, outo_h.at[pl.ds(cid * NACC + tb, RPT)])


def _tc(body, n_out):
    shapes = []
    for s in n_out:
        shapes.append(jax.ShapeDtypeStruct(s, _f32))
    return pl.pallas_call(body, out_shape=shapes)


def _deg_body(dp_ref, op_ref, dinv_o, dout_o):
    deg = dp_ref[0] + dp_ref[1] + 1.0
    dinv_o[...] = lax.rsqrt(jnp.maximum(deg, 1e-12))
    dout_o[...] = op_ref[0] + op_ref[1]


def _mm_body(x_ref, w_ref, b_ref, dinv_ref, hh_o, sp_o):
    h = jnp.dot(x_ref[...], w_ref[...], preferred_element_type=_f32)
    dv = dinv_ref[...][:, None]
    hh_o[...] = dv * h
    sp_o[...] = b_ref[...][None, :] + dv * dv * h


def _h1_body(sp_ref, dinv_ref, rp_ref, h1_o, n_o):
    h1 = sp_ref[...] + dinv_ref[...][:, None] * (rp_ref[0] + rp_ref[1])
    h1_o[...] = h1
    n_o[...] = jnp.sum(h1 * h1, axis=1)


def _fin_body(sp_ref, dinv_ref, rp_ref, o_ref):
    o_ref[...] = sp_ref[...] + dinv_ref[...][:, None] * (rp_ref[0] + rp_ref[1])


def _coef_body(h1_ref, n_ref, dout_ref, s1p_ref, t1p_ref, a_o, ah_o, s1_o):
    h1 = h1_ref[...]
    s1 = s1p_ref[0] + s1p_ref[1]
    s1_o[...] = s1
    t1 = t1p_ref[0] + t1p_ref[1]
    e = 0.5 * dout_ref[...] * n_ref[...] - jnp.sum(h1 * s1, axis=1) + 0.5 * t1
    valid = lax.broadcasted_iota(jnp.int32, (NACC,), 0) < N_NODES
    m = jnp.max(jnp.where(valid, e, -jnp.inf))
    c = jnp.maximum(m, 1e-9)
    et = e / c
    z = jnp.where(valid, -et / TEMP, -jnp.inf)
    p = jnp.exp(z - jnp.max(z))
    p = p / jnp.sum(p)
    g = -(jnp.log(p + 1e-12) + p / (p + 1e-12))
    dz = p * (g - jnp.sum(g * p))
    q = -dz / TEMP
    mask = jnp.where(valid & (e == m), 1.0, 0.0)
    nt = jnp.maximum(jnp.sum(mask), 1.0)
    dcdm = jnp.where(m > 1e-9, 1.0, jnp.where(m == 1e-9, 0.5, 0.0))
    beta = dcdm * (-jnp.sum(q * et) / c)
    a = jnp.where(valid, q / c + beta * mask / nt, 0.0)
    a_o[...] = a
    ah_o[...] = a[:, None] * h1


def _grad_body(h1_ref, a_ref, dout_ref, s1_ref, s2p_ref, s3p_ref, o_ref):
    h1 = h1_ref[...]
    s2 = s2p_ref[0] + s2p_ref[1]
    s3 = (s3p_ref[0] + s3p_ref[1])[:, None]
    av = a_ref[...][:, None]
    grd = av * (dout_ref[...][:, None] * h1 - s1_ref[...]) - s2 + s3 * h1
    o_ref[...] = jnp.maximum(h1 + WGT * grd, 0.0)


def kernel(x, edge_index, W1, b1, W2, b2, W_out, b_out):
    src = edge_index[0]
    dst = edge_index[1]
    npad = E_PAD - N_EDGES
    ar = jnp.arange(npad, dtype=jnp.int32)
    pad_g = ar % N_NODES
    pad_s = N_NODES + ar % (NACC - N_NODES)
    src_g = jnp.concatenate([src, pad_g]).reshape(NW, NCHUNK, CH)
    src_s = jnp.concatenate([src, pad_s]).reshape(NW, NCHUNK, CH)
    dst_g = jnp.concatenate([dst, pad_g]).reshape(NW, NCHUNK, CH)
    dst_s = jnp.concatenate([dst, pad_s]).reshape(NW, NCHUNK, CH)
    z2 = jnp.zeros((NACC, D), _f32)
    xp = jnp.pad(x, ((0, NACC - N_NODES), (0, 0)))

    degp, degop = _sc_deg(dst_s, src_s)
    dinv, deg_out = _tc(_deg_body, [(NACC,), (NACC,)])(
        degp.reshape(NC, NACC), degop.reshape(NC, NACC))

    h = xp
    for wgt, bias in ((W1, b1), (W2, b2)):
        hh, sp = _tc(_mm_body, [(NACC, D), (NACC, D)])(h, wgt, bias, dinv)
        rows_a = _sc_rows(hh, src_g, dst_s, z2)[0].reshape(NC, NACC, D)
        h1, n = _tc(_h1_body, [(NACC, D), (NACC,)])(sp, dinv, rows_a)
        rows_b, t1p = _sc_rows_scal(h1, n, dst_g, src_s, z2)
        a, ah, s1 = _tc(_coef_body, [(NACC,), (NACC, D), (NACC, D)])(
            h1, n, deg_out, rows_b.reshape(NC, NACC, D), t1p.reshape(NC, NACC))
        rows_c, s3p = _sc_rows_scal(ah, a, src_g, dst_s, z2)
        h = _tc(_grad_body, [(NACC, D)])(
            h1, a, deg_out, s1, rows_c.reshape(NC, NACC, D),
            s3p.reshape(NC, NACC))[0]

    hh, sp = _tc(_mm_body, [(NACC, D), (NACC, D)])(h, W_out, b_out, dinv)
    rows_f = _sc_rows(hh, src_g, dst_s, z2)[0].reshape(NC, NACC, D)
    emb = _tc(_fin_body, [(NACC, D)])(sp, dinv, rows_f)[0]
    return emb[:N_NODES]


# speedup vs baseline: 1.5657x; 1.0114x over previous
import functools

import jax
import jax.numpy as jnp
from jax import lax
from jax.experimental import pallas as pl
from jax.experimental.pallas import tpu as pltpu
from jax.experimental.pallas import tpu_sc as plsc

N_NODES = 10000
D = 128
N_EDGES = 320000
NC = 2
NS = 16
NW = NC * NS
CH = 128
NCHUNK = 80
EPW = NCHUNK * CH
E_PAD = NW * EPW
NACC = 10240
RPT = NACC // NS
TEMP = 10.0
WGT = 1.0

_mesh = plsc.VectorSubcoreMesh(core_axis_name="c", subcore_axis_name="s")
_f32 = jnp.float32


@functools.partial(
    pl.kernel,
    out_type=[jax.ShapeDtypeStruct((NC * NACC, D), _f32)],
    mesh=_mesh,
    scratch_types=[
        pltpu.VMEM((NCHUNK, CH), jnp.int32),
        pltpu.VMEM((2, 1, CH), jnp.int32),
        pltpu.VMEM((2, CH, D), _f32),
        pltpu.VMEM((16, D), _f32),
        pltpu.VMEM_SHARED((NACC, D), _f32),
        pltpu.SemaphoreType.DMA,
        pltpu.SemaphoreType.DMA,
        pltpu.SemaphoreType.DMA,
    ],
)
def _sc_rows(table_h, gidx_h, sidx_h, out_h,
             gidx_v, sidx_v, rows_v, zrow_v, acc_r, sem_g, sem_i0, sem_i1):
    cid = lax.axis_index("c")
    sid = lax.axis_index("s")
    wid = sid * NC + cid
    tb = sid * RPT
    for r in range(16):
        for k in range(D // 16):
            zrow_v[r, pl.ds(k * 16, 16)] = jnp.zeros((16,), _f32)

    def zstep(j, carry):
        pltpu.sync_copy(zrow_v, acc_r.at[pl.ds(tb + j * 16, 16)])
        return carry

    lax.fori_loop(0, RPT // 16, zstep, 0)
    pltpu.sync_copy(gidx_h.at[wid], gidx_v)
    plsc.subcore_barrier()
    pltpu.async_copy(table_h.at[gidx_v.at[0]], rows_v.at[0], sem_g)
    pltpu.async_copy(sidx_h.at[wid, pl.ds(0, 1)], sidx_v.at[0], sem_i0)
    pltpu.async_copy(sidx_h.at[wid, pl.ds(1, 1)], sidx_v.at[1], sem_i1)
    isems = (sem_i0, sem_i1)

    def step(g, carry):
        for b in range(2):
            ci = 2 * g + b
            pltpu.make_async_copy(
                table_h.at[gidx_v.at[ci]], rows_v.at[b], sem_g).wait()
            nx = jnp.minimum(ci + 1, NCHUNK - 1)
            pltpu.async_copy(table_h.at[gidx_v.at[nx]], rows_v.at[1 - b], sem_g)
            pltpu.make_async_copy(
                sidx_h.at[wid, pl.ds(ci, 1)], sidx_v.at[b], isems[b]).wait()
            pltpu.sync_copy(rows_v.at[b], acc_r.at[sidx_v.at[b, 0]], add=True)
            nx2 = jnp.minimum(ci + 2, NCHUNK - 1)
            pltpu.async_copy(sidx_h.at[wid, pl.ds(nx2, 1)], sidx_v.at[b],
                             isems[b])
        return carry

    lax.fori_loop(0, NCHUNK // 2, step, 0)
    pltpu.make_async_copy(
        table_h.at[gidx_v.at[NCHUNK - 1]], rows_v.at[0], sem_g).wait()
    pltpu.make_async_copy(
        sidx_h.at[wid, pl.ds(0, 1)], sidx_v.at[0], sem_i0).wait()
    pltpu.make_async_copy(
        sidx_h.at[wid, pl.ds(0, 1)], sidx_v.at[1], sem_i1).wait()
    plsc.subcore_barrier()
    pltpu.sync_copy(acc_r.at[pl.ds(tb, RPT)],
                    out_h.at[pl.ds(cid * NACC + tb, RPT)])


@functools.partial(
    pl.kernel,
    out_type=[jax.ShapeDtypeStruct((NC * NACC, D), _f32),
              jax.ShapeDtypeStruct((NC * NACC,), _f32)],
    mesh=_mesh,
    scratch_types=[
        pltpu.VMEM((NCHUNK, CH), jnp.int32),
        pltpu.VMEM((2, 1, CH), jnp.int32),
        pltpu.VMEM((2, CH, D), _f32),
        pltpu.VMEM((2, CH), _f32),
        pltpu.VMEM((16, D), _f32),
        pltpu.VMEM((RPT,), _f32),
        pltpu.VMEM_SHARED((NACC, D), _f32),
        pltpu.VMEM_SHARED((NACC,), _f32),
        pltpu.SemaphoreType.DMA,
        pltpu.SemaphoreType.DMA,
        pltpu.SemaphoreType.DMA,
        pltpu.SemaphoreType.DMA,
    ],
)
def _sc_rows_scal(table_h, tab1_h, gidx_h, sidx_h, out_h, out1_h,
                  gidx_v, sidx_v, rows_v, scal_v, zrow_v, bnc_v, acc_r, acc_s,
                  sem_r, sem_s, sem_i0, sem_i1):
    cid = lax.axis_index("c")
    sid = lax.axis_index("s")
    wid = sid * NC + cid
    tb = sid * RPT
    for r in range(16):
        for k in range(D // 16):
            zrow_v[r, pl.ds(k * 16, 16)] = jnp.zeros((16,), _f32)

    def zstep(j, carry):
        pltpu.sync_copy(zrow_v, acc_r.at[pl.ds(tb + j * 16, 16)])
        return carry

    lax.fori_loop(0, RPT // 16, zstep, 0)
    for k in range(RPT // 16):
        bnc_v[pl.ds(k * 16, 16)] = jnp.zeros((16,), _f32)
    pltpu.sync_copy(bnc_v, acc_s.at[pl.ds(tb, RPT)])
    pltpu.sync_copy(gidx_h.at[wid], gidx_v)
    plsc.subcore_barrier()
    pltpu.async_copy(table_h.at[gidx_v.at[0]], rows_v.at[0], sem_r)
    pltpu.async_copy(tab1_h.at[gidx_v.at[0]], scal_v.at[0], sem_s)
    pltpu.async_copy(sidx_h.at[wid, pl.ds(0, 1)], sidx_v.at[0], sem_i0)
    pltpu.async_copy(sidx_h.at[wid, pl.ds(1, 1)], sidx_v.at[1], sem_i1)
    isems = (sem_i0, sem_i1)

    def step(g, carry):
        for b in range(2):
            ci = 2 * g + b
            pltpu.make_async_copy(
                table_h.at[gidx_v.at[ci]], rows_v.at[b], sem_r).wait()
            pltpu.make_async_copy(
                tab1_h.at[gidx_v.at[ci]], scal_v.at[b], sem_s).wait()
            nx = jnp.minimum(ci + 1, NCHUNK - 1)
            pltpu.async_copy(table_h.at[gidx_v.at[nx]], rows_v.at[1 - b], sem_r)
            pltpu.async_copy(tab1_h.at[gidx_v.at[nx]], scal_v.at[1 - b], sem_s)
            pltpu.make_async_copy(
                sidx_h.at[wid, pl.ds(ci, 1)], sidx_v.at[b], isems[b]).wait()
            pltpu.sync_copy(rows_v.at[b], acc_r.at[sidx_v.at[b, 0]], add=True)
            pltpu.sync_copy(scal_v.at[b], acc_s.at[sidx_v.at[b, 0]], add=True)
            nx2 = jnp.minimum(ci + 2, NCHUNK - 1)
            pltpu.async_copy(sidx_h.at[wid, pl.ds(nx2, 1)], sidx_v.at[b],
                             isems[b])
        return carry

    lax.fori_loop(0, NCHUNK // 2, step, 0)
    pltpu.make_async_copy(
        table_h.at[gidx_v.at[NCHUNK - 1]], rows_v.at[0], sem_r).wait()
    pltpu.make_async_copy(
        tab1_h.at[gidx_v.at[NCHUNK - 1]], scal_v.at[0], sem_s).wait()
    pltpu.make_async_copy(
        sidx_h.at[wid, pl.ds(0, 1)], sidx_v.at[0], sem_i0).wait()
    pltpu.make_async_copy(
        sidx_h.at[wid, pl.ds(0, 1)], sidx_v.at[1], sem_i1).wait()
    plsc.subcore_barrier()
    pltpu.sync_copy(acc_r.at[pl.ds(tb, RPT)],
                    out_h.at[pl.ds(cid * NACC + tb, RPT)])
    pltpu.sync_copy(acc_s.at[pl.ds(tb, RPT)], bnc_v)
    pltpu.sync_copy(bnc_v, out1_h.at[pl.ds(cid * NACC + tb, RPT)])


@functools.partial(
    pl.kernel,
    out_type=[jax.ShapeDtypeStruct((NC * NACC,), _f32),
              jax.ShapeDtypeStruct((NC * NACC,), _f32)],
    mesh=_mesh,
    scratch_types=[
        pltpu.VMEM((NCHUNK, CH), jnp.int32),
        pltpu.VMEM((NCHUNK, CH), jnp.int32),
        pltpu.VMEM((CH,), _f32),
        pltpu.VMEM((RPT,), _f32),
        pltpu.VMEM_SHARED((NACC,), _f32),
        pltpu.VMEM_SHARED((NACC,), _f32),
        pltpu.SemaphoreType.DMA,
        pltpu.SemaphoreType.DMA,
    ],
)
def _sc_deg(didx_h, oidx_h, outd_h, outo_h,
            didx_v, oidx_v, ones_v, bnc_v, acc_d, acc_o, sem_d, sem_o):
    cid = lax.axis_index("c")
    sid = lax.axis_index("s")
    wid = sid * NC + cid
    tb = sid * RPT
    for k in range(RPT // 16):
        bnc_v[pl.ds(k * 16, 16)] = jnp.zeros((16,), _f32)
    pltpu.sync_copy(bnc_v, acc_d.at[pl.ds(tb, RPT)])
    pltpu.sync_copy(bnc_v, acc_o.at[pl.ds(tb, RPT)])
    pltpu.sync_copy(didx_h.at[wid], didx_v)
    pltpu.sync_copy(oidx_h.at[wid], oidx_v)
    for k in range(CH // 16):
        ones_v[pl.ds(k * 16, 16)] = jnp.full((16,), 1.0, _f32)
    plsc.subcore_barrier()

    def step(ci, carry):
        pltpu.async_copy(ones_v, acc_d.at[didx_v.at[ci]], sem_d, add=True)
        pltpu.async_copy(ones_v, acc_o.at[oidx_v.at[ci]], sem_o, add=True)
        return carry

    lax.fori_loop(0, NCHUNK, step, 0)

    def drain(ci, carry):
        pltpu.make_async_copy(ones_v, acc_d.at[didx_v.at[ci]], sem_d).wait()
        pltpu.make_async_copy(ones_v, acc_o.at[oidx_v.at[ci]], sem_o).wait()
        return carry

    lax.fori_loop(0, NCHUNK, drain, 0)
    plsc.subcore_barrier()
    pltpu.sync_copy(acc_d.at[pl.ds(tb, RPT)], bnc_v)
    pltpu.sync_copy(bnc_v, outd_h.at[pl.ds(cid * NACC + tb, RPT)])
    pltpu.sync_copy(acc_o.at[pl.ds(tb, RPT)], bnc_v)
    pltpu.sync_copy(bnc_v, outo_h.at[pl.ds(cid * NACC + tb, RPT)])


def _tc(body, n_out):
    shapes = []
    for s in n_out:
        shapes.append(jax.ShapeDtypeStruct(s, _f32))
    return pl.pallas_call(body, out_shape=shapes)


def _deg_body(dp_ref, op_ref, dinv_o, dout_o):
    deg = dp_ref[0] + dp_ref[1] + 1.0
    dinv_o[...] = lax.rsqrt(jnp.maximum(deg, 1e-12))
    dout_o[...] = op_ref[0] + op_ref[1]


def _mm_body(x_ref, w_ref, b_ref, dinv_ref, hh_o, sp_o):
    h = jnp.dot(x_ref[...], w_ref[...], preferred_element_type=_f32)
    dv = dinv_ref[...][:, None]
    hh_o[...] = dv * h
    sp_o[...] = b_ref[...][None, :] + dv * dv * h


def _h1_body(sp_ref, dinv_ref, rp_ref, h1_o, n_o):
    h1 = sp_ref[...] + dinv_ref[...][:, None] * (rp_ref[0] + rp_ref[1])
    h1_o[...] = h1
    n_o[...] = jnp.sum(h1 * h1, axis=1)


def _fin_body(sp_ref, dinv_ref, rp_ref, o_ref):
    o_ref[...] = sp_ref[...] + dinv_ref[...][:, None] * (rp_ref[0] + rp_ref[1])


def _coef_body(h1_ref, n_ref, dout_ref, s1p_ref, t1p_ref, a_o, ah_o, s1_o):
    h1 = h1_ref[...]
    s1 = s1p_ref[0] + s1p_ref[1]
    s1_o[...] = s1
    t1 = t1p_ref[0] + t1p_ref[1]
    e = 0.5 * dout_ref[...] * n_ref[...] - jnp.sum(h1 * s1, axis=1) + 0.5 * t1
    valid = lax.broadcasted_iota(jnp.int32, (NACC,), 0) < N_NODES
    m = jnp.max(jnp.where(valid, e, -jnp.inf))
    c = jnp.maximum(m, 1e-9)
    et = e / c
    z = jnp.where(valid, -et / TEMP, -jnp.inf)
    p = jnp.exp(z - jnp.max(z))
    p = p / jnp.sum(p)
    g = -(jnp.log(p + 1e-12) + p / (p + 1e-12))
    dz = p * (g - jnp.sum(g * p))
    q = -dz / TEMP
    mask = jnp.where(valid & (e == m), 1.0, 0.0)
    nt = jnp.maximum(jnp.sum(mask), 1.0)
    dcdm = jnp.where(m > 1e-9, 1.0, jnp.where(m == 1e-9, 0.5, 0.0))
    beta = dcdm * (-jnp.sum(q * et) / c)
    a = jnp.where(valid, q / c + beta * mask / nt, 0.0)
    a_o[...] = a
    ah_o[...] = a[:, None] * h1


def _grad_body(h1_ref, a_ref, dout_ref, s1_ref, s2p_ref, s3p_ref, o_ref):
    h1 = h1_ref[...]
    s2 = s2p_ref[0] + s2p_ref[1]
    s3 = (s3p_ref[0] + s3p_ref[1])[:, None]
    av = a_ref[...][:, None]
    grd = av * (dout_ref[...][:, None] * h1 - s1_ref[...]) - s2 + s3 * h1
    o_ref[...] = jnp.maximum(h1 + WGT * grd, 0.0)


def kernel(x, edge_index, W1, b1, W2, b2, W_out, b_out):
    src = edge_index[0]
    dst = edge_index[1]
    npad = E_PAD - N_EDGES
    ar = jnp.arange(npad, dtype=jnp.int32)
    pad_g = ar % N_NODES
    pad_s = N_NODES + ar % (NACC - N_NODES)
    src_g = jnp.concatenate([src, pad_g]).reshape(NW, NCHUNK, CH)
    src_s = jnp.concatenate([src, pad_s]).reshape(NW, NCHUNK, CH)
    dst_g = jnp.concatenate([dst, pad_g]).reshape(NW, NCHUNK, CH)
    dst_s = jnp.concatenate([dst, pad_s]).reshape(NW, NCHUNK, CH)
    xp = jnp.pad(x, ((0, NACC - N_NODES), (0, 0)))

    degp, degop = _sc_deg(dst_s, src_s)
    dinv, deg_out = _tc(_deg_body, [(NACC,), (NACC,)])(
        degp.reshape(NC, NACC), degop.reshape(NC, NACC))

    h = xp
    for wgt, bias in ((W1, b1), (W2, b2)):
        hh, sp = _tc(_mm_body, [(NACC, D), (NACC, D)])(h, wgt, bias, dinv)
        rows_a = _sc_rows(hh, src_g, dst_s)[0].reshape(NC, NACC, D)
        h1, n = _tc(_h1_body, [(NACC, D), (NACC,)])(sp, dinv, rows_a)
        rows_b, t1p = _sc_rows_scal(h1, n, dst_g, src_s)
        a, ah, s1 = _tc(_coef_body, [(NACC,), (NACC, D), (NACC, D)])(
            h1, n, deg_out, rows_b.reshape(NC, NACC, D), t1p.reshape(NC, NACC))
        rows_c, s3p = _sc_rows_scal(ah, a, src_g, dst_s)
        h = _tc(_grad_body, [(NACC, D)])(
            h1, a, deg_out, s1, rows_c.reshape(NC, NACC, D),
            s3p.reshape(NC, NACC))[0]

    hh, sp = _tc(_mm_body, [(NACC, D), (NACC, D)])(h, W_out, b_out, dinv)
    rows_f = _sc_rows(hh, src_g, dst_s)[0].reshape(NC, NACC, D)
    emb = _tc(_fin_body, [(NACC, D)])(sp, dinv, rows_f)[0]
    return emb[:N_NODES]


# speedup vs baseline: 1.5883x; 1.0144x over previous
import functools

import jax
import jax.numpy as jnp
from jax import lax
from jax.experimental import pallas as pl
from jax.experimental.pallas import tpu as pltpu
from jax.experimental.pallas import tpu_sc as plsc

N_NODES = 10000
D = 128
N_EDGES = 320000
NC = 2
NS = 16
NW = NC * NS
CH = 128
NCHUNK = 80
EPW = NCHUNK * CH
E_PAD = NW * EPW
NACC = 10240
RPT = NACC // NS
TEMP = 10.0
WGT = 1.0

_mesh = plsc.VectorSubcoreMesh(core_axis_name="c", subcore_axis_name="s")
_f32 = jnp.float32


@functools.partial(
    pl.kernel,
    out_type=[jax.ShapeDtypeStruct((NC * NACC, D), _f32)],
    mesh=_mesh,
    scratch_types=[
        pltpu.VMEM((NCHUNK, CH), jnp.int32),
        pltpu.VMEM((2, 1, CH), jnp.int32),
        pltpu.VMEM((2, CH, D), _f32),
        pltpu.VMEM((16, D), _f32),
        pltpu.VMEM_SHARED((NACC, D), _f32),
        pltpu.SemaphoreType.DMA,
        pltpu.SemaphoreType.DMA,
        pltpu.SemaphoreType.DMA,
    ],
)
def _sc_rows(table_h, gidx_h, sidx_h, out_h,
             gidx_v, sidx_v, rows_v, zrow_v, acc_r, sem_g, sem_i0, sem_i1):
    cid = lax.axis_index("c")
    sid = lax.axis_index("s")
    wid = sid * NC + cid
    tb = sid * RPT
    for r in range(16):
        for k in range(D // 16):
            zrow_v[r, pl.ds(k * 16, 16)] = jnp.zeros((16,), _f32)

    def zstep(j, carry):
        pltpu.async_copy(zrow_v, acc_r.at[pl.ds(tb + j * 16, 16)], sem_g)
        return carry

    lax.fori_loop(0, RPT // 16, zstep, 0)
    pltpu.sync_copy(gidx_h.at[wid], gidx_v)

    def zdrain(j, carry):
        pltpu.make_async_copy(
            zrow_v, acc_r.at[pl.ds(tb + j * 16, 16)], sem_g).wait()
        return carry

    lax.fori_loop(0, RPT // 16, zdrain, 0)
    plsc.subcore_barrier()
    pltpu.async_copy(table_h.at[gidx_v.at[0]], rows_v.at[0], sem_g)
    pltpu.async_copy(sidx_h.at[wid, pl.ds(0, 1)], sidx_v.at[0], sem_i0)
    pltpu.async_copy(sidx_h.at[wid, pl.ds(1, 1)], sidx_v.at[1], sem_i1)
    isems = (sem_i0, sem_i1)

    def step(g, carry):
        for b in range(2):
            ci = 2 * g + b
            pltpu.make_async_copy(
                table_h.at[gidx_v.at[ci]], rows_v.at[b], sem_g).wait()
            nx = jnp.minimum(ci + 1, NCHUNK - 1)
            pltpu.async_copy(table_h.at[gidx_v.at[nx]], rows_v.at[1 - b], sem_g)
            pltpu.make_async_copy(
                sidx_h.at[wid, pl.ds(ci, 1)], sidx_v.at[b], isems[b]).wait()
            pltpu.sync_copy(rows_v.at[b], acc_r.at[sidx_v.at[b, 0]], add=True)
            nx2 = jnp.minimum(ci + 2, NCHUNK - 1)
            pltpu.async_copy(sidx_h.at[wid, pl.ds(nx2, 1)], sidx_v.at[b],
                             isems[b])
        return carry

    lax.fori_loop(0, NCHUNK // 2, step, 0)
    pltpu.make_async_copy(
        table_h.at[gidx_v.at[NCHUNK - 1]], rows_v.at[0], sem_g).wait()
    pltpu.make_async_copy(
        sidx_h.at[wid, pl.ds(0, 1)], sidx_v.at[0], sem_i0).wait()
    pltpu.make_async_copy(
        sidx_h.at[wid, pl.ds(0, 1)], sidx_v.at[1], sem_i1).wait()
    plsc.subcore_barrier()
    pltpu.sync_copy(acc_r.at[pl.ds(tb, RPT)],
                    out_h.at[pl.ds(cid * NACC + tb, RPT)])


@functools.partial(
    pl.kernel,
    out_type=[jax.ShapeDtypeStruct((NC * NACC, D), _f32),
              jax.ShapeDtypeStruct((NC * NACC,), _f32)],
    mesh=_mesh,
    scratch_types=[
        pltpu.VMEM((NCHUNK, CH), jnp.int32),
        pltpu.VMEM((2, 1, CH), jnp.int32),
        pltpu.VMEM((2, CH, D), _f32),
        pltpu.VMEM((2, CH), _f32),
        pltpu.VMEM((16, D), _f32),
        pltpu.VMEM((RPT,), _f32),
        pltpu.VMEM_SHARED((NACC, D), _f32),
        pltpu.VMEM_SHARED((NACC,), _f32),
        pltpu.SemaphoreType.DMA,
        pltpu.SemaphoreType.DMA,
        pltpu.SemaphoreType.DMA,
        pltpu.SemaphoreType.DMA,
    ],
)
def _sc_rows_scal(table_h, tab1_h, gidx_h, sidx_h, out_h, out1_h,
                  gidx_v, sidx_v, rows_v, scal_v, zrow_v, bnc_v, acc_r, acc_s,
                  sem_r, sem_s, sem_i0, sem_i1):
    cid = lax.axis_index("c")
    sid = lax.axis_index("s")
    wid = sid * NC + cid
    tb = sid * RPT
    for r in range(16):
        for k in range(D // 16):
            zrow_v[r, pl.ds(k * 16, 16)] = jnp.zeros((16,), _f32)

    def zstep(j, carry):
        pltpu.async_copy(zrow_v, acc_r.at[pl.ds(tb + j * 16, 16)], sem_r)
        return carry

    lax.fori_loop(0, RPT // 16, zstep, 0)
    for k in range(RPT // 16):
        bnc_v[pl.ds(k * 16, 16)] = jnp.zeros((16,), _f32)
    pltpu.sync_copy(bnc_v, acc_s.at[pl.ds(tb, RPT)])
    pltpu.sync_copy(gidx_h.at[wid], gidx_v)

    def zdrain(j, carry):
        pltpu.make_async_copy(
            zrow_v, acc_r.at[pl.ds(tb + j * 16, 16)], sem_r).wait()
        return carry

    lax.fori_loop(0, RPT // 16, zdrain, 0)
    plsc.subcore_barrier()
    pltpu.async_copy(table_h.at[gidx_v.at[0]], rows_v.at[0], sem_r)
    pltpu.async_copy(tab1_h.at[gidx_v.at[0]], scal_v.at[0], sem_s)
    pltpu.async_copy(sidx_h.at[wid, pl.ds(0, 1)], sidx_v.at[0], sem_i0)
    pltpu.async_copy(sidx_h.at[wid, pl.ds(1, 1)], sidx_v.at[1], sem_i1)
    isems = (sem_i0, sem_i1)

    def step(g, carry):
        for b in range(2):
            ci = 2 * g + b
            pltpu.make_async_copy(
                table_h.at[gidx_v.at[ci]], rows_v.at[b], sem_r).wait()
            pltpu.make_async_copy(
                tab1_h.at[gidx_v.at[ci]], scal_v.at[b], sem_s).wait()
            nx = jnp.minimum(ci + 1, NCHUNK - 1)
            pltpu.async_copy(table_h.at[gidx_v.at[nx]], rows_v.at[1 - b], sem_r)
            pltpu.async_copy(tab1_h.at[gidx_v.at[nx]], scal_v.at[1 - b], sem_s)
            pltpu.make_async_copy(
                sidx_h.at[wid, pl.ds(ci, 1)], sidx_v.at[b], isems[b]).wait()
            pltpu.sync_copy(rows_v.at[b], acc_r.at[sidx_v.at[b, 0]], add=True)
            pltpu.sync_copy(scal_v.at[b], acc_s.at[sidx_v.at[b, 0]], add=True)
            nx2 = jnp.minimum(ci + 2, NCHUNK - 1)
            pltpu.async_copy(sidx_h.at[wid, pl.ds(nx2, 1)], sidx_v.at[b],
                             isems[b])
        return carry

    lax.fori_loop(0, NCHUNK // 2, step, 0)
    pltpu.make_async_copy(
        table_h.at[gidx_v.at[NCHUNK - 1]], rows_v.at[0], sem_r).wait()
    pltpu.make_async_copy(
        tab1_h.at[gidx_v.at[NCHUNK - 1]], scal_v.at[0], sem_s).wait()
    pltpu.make_async_copy(
        sidx_h.at[wid, pl.ds(0, 1)], sidx_v.at[0], sem_i0).wait()
    pltpu.make_async_copy(
        sidx_h.at[wid, pl.ds(0, 1)], sidx_v.at[1], sem_i1).wait()
    plsc.subcore_barrier()
    pltpu.sync_copy(acc_r.at[pl.ds(tb, RPT)],
                    out_h.at[pl.ds(cid * NACC + tb, RPT)])
    pltpu.sync_copy(acc_s.at[pl.ds(tb, RPT)], bnc_v)
    pltpu.sync_copy(bnc_v, out1_h.at[pl.ds(cid * NACC + tb, RPT)])


@functools.partial(
    pl.kernel,
    out_type=[jax.ShapeDtypeStruct((NC * NACC,), _f32),
              jax.ShapeDtypeStruct((NC * NACC,), _f32)],
    mesh=_mesh,
    scratch_types=[
        pltpu.VMEM((NCHUNK, CH), jnp.int32),
        pltpu.VMEM((NCHUNK, CH), jnp.int32),
        pltpu.VMEM((CH,), _f32),
        pltpu.VMEM((RPT,), _f32),
        pltpu.VMEM_SHARED((NACC,), _f32),
        pltpu.VMEM_SHARED((NACC,), _f32),
        pltpu.SemaphoreType.DMA,
        pltpu.SemaphoreType.DMA,
    ],
)
def _sc_deg(didx_h, oidx_h, outd_h, outo_h,
            didx_v, oidx_v, ones_v, bnc_v, acc_d, acc_o, sem_d, sem_o):
    cid = lax.axis_index("c")
    sid = lax.axis_index("s")
    wid = sid * NC + cid
    tb = sid * RPT
    for k in range(RPT // 16):
        bnc_v[pl.ds(k * 16, 16)] = jnp.zeros((16,), _f32)
    pltpu.sync_copy(bnc_v, acc_d.at[pl.ds(tb, RPT)])
    pltpu.sync_copy(bnc_v, acc_o.at[pl.ds(tb, RPT)])
    pltpu.sync_copy(didx_h.at[wid], didx_v)
    pltpu.sync_copy(oidx_h.at[wid], oidx_v)
    for k in range(CH // 16):
        ones_v[pl.ds(k * 16, 16)] = jnp.full((16,), 1.0, _f32)
    plsc.subcore_barrier()

    def step(ci, carry):
        pltpu.async_copy(ones_v, acc_d.at[didx_v.at[ci]], sem_d, add=True)
        pltpu.async_copy(ones_v, acc_o.at[oidx_v.at[ci]], sem_o, add=True)
        return carry

    lax.fori_loop(0, NCHUNK, step, 0)

    def drain(ci, carry):
        pltpu.make_async_copy(ones_v, acc_d.at[didx_v.at[ci]], sem_d).wait()
        pltpu.make_async_copy(ones_v, acc_o.at[oidx_v.at[ci]], sem_o).wait()
        return carry

    lax.fori_loop(0, NCHUNK, drain, 0)
    plsc.subcore_barrier()
    pltpu.sync_copy(acc_d.at[pl.ds(tb, RPT)], bnc_v)
    pltpu.sync_copy(bnc_v, outd_h.at[pl.ds(cid * NACC + tb, RPT)])
    pltpu.sync_copy(acc_o.at[pl.ds(tb, RPT)], bnc_v)
    pltpu.sync_copy(bnc_v, outo_h.at[pl.ds(cid * NACC + tb, RPT)])


def _tc(body, n_out):
    shapes = []
    for s in n_out:
        shapes.append(jax.ShapeDtypeStruct(s, _f32))
    return pl.pallas_call(body, out_shape=shapes)


def _deg_body(dp_ref, op_ref, dinv_o, dout_o):
    deg = dp_ref[0] + dp_ref[1] + 1.0
    dinv_o[...] = lax.rsqrt(jnp.maximum(deg, 1e-12))
    dout_o[...] = op_ref[0] + op_ref[1]


def _mm_body(x_ref, w_ref, b_ref, dinv_ref, hh_o, sp_o):
    h = jnp.dot(x_ref[...], w_ref[...], preferred_element_type=_f32)
    dv = dinv_ref[...][:, None]
    hh_o[...] = dv * h
    sp_o[...] = b_ref[...][None, :] + dv * dv * h


def _h1_body(sp_ref, dinv_ref, rp_ref, h1_o, n_o):
    h1 = sp_ref[...] + dinv_ref[...][:, None] * (rp_ref[0] + rp_ref[1])
    h1_o[...] = h1
    n_o[...] = jnp.sum(h1 * h1, axis=1)


def _fin_body(sp_ref, dinv_ref, rp_ref, o_ref):
    o_ref[...] = sp_ref[...] + dinv_ref[...][:, None] * (rp_ref[0] + rp_ref[1])


def _coef_body(h1_ref, n_ref, dout_ref, s1p_ref, t1p_ref, a_o, ah_o, s1_o):
    h1 = h1_ref[...]
    s1 = s1p_ref[0] + s1p_ref[1]
    s1_o[...] = s1
    t1 = t1p_ref[0] + t1p_ref[1]
    e = 0.5 * dout_ref[...] * n_ref[...] - jnp.sum(h1 * s1, axis=1) + 0.5 * t1
    valid = lax.broadcasted_iota(jnp.int32, (NACC,), 0) < N_NODES
    m = jnp.max(jnp.where(valid, e, -jnp.inf))
    c = jnp.maximum(m, 1e-9)
    et = e / c
    z = jnp.where(valid, -et / TEMP, -jnp.inf)
    p = jnp.exp(z - jnp.max(z))
    p = p / jnp.sum(p)
    g = -(jnp.log(p + 1e-12) + p / (p + 1e-12))
    dz = p * (g - jnp.sum(g * p))
    q = -dz / TEMP
    mask = jnp.where(valid & (e == m), 1.0, 0.0)
    nt = jnp.maximum(jnp.sum(mask), 1.0)
    dcdm = jnp.where(m > 1e-9, 1.0, jnp.where(m == 1e-9, 0.5, 0.0))
    beta = dcdm * (-jnp.sum(q * et) / c)
    a = jnp.where(valid, q / c + beta * mask / nt, 0.0)
    a_o[...] = a
    ah_o[...] = a[:, None] * h1


def _grad_body(h1_ref, a_ref, dout_ref, s1_ref, s2p_ref, s3p_ref, o_ref):
    h1 = h1_ref[...]
    s2 = s2p_ref[0] + s2p_ref[1]
    s3 = (s3p_ref[0] + s3p_ref[1])[:, None]
    av = a_ref[...][:, None]
    grd = av * (dout_ref[...][:, None] * h1 - s1_ref[...]) - s2 + s3 * h1
    o_ref[...] = jnp.maximum(h1 + WGT * grd, 0.0)


def kernel(x, edge_index, W1, b1, W2, b2, W_out, b_out):
    src = edge_index[0]
    dst = edge_index[1]
    npad = E_PAD - N_EDGES
    ar = jnp.arange(npad, dtype=jnp.int32)
    pad_g = ar % N_NODES
    pad_s = N_NODES + ar % (NACC - N_NODES)
    src_g = jnp.concatenate([src, pad_g]).reshape(NW, NCHUNK, CH)
    src_s = jnp.concatenate([src, pad_s]).reshape(NW, NCHUNK, CH)
    dst_g = jnp.concatenate([dst, pad_g]).reshape(NW, NCHUNK, CH)
    dst_s = jnp.concatenate([dst, pad_s]).reshape(NW, NCHUNK, CH)
    xp = jnp.pad(x, ((0, NACC - N_NODES), (0, 0)))

    degp, degop = _sc_deg(dst_s, src_s)
    dinv, deg_out = _tc(_deg_body, [(NACC,), (NACC,)])(
        degp.reshape(NC, NACC), degop.reshape(NC, NACC))

    h = xp
    for wgt, bias in ((W1, b1), (W2, b2)):
        hh, sp = _tc(_mm_body, [(NACC, D), (NACC, D)])(h, wgt, bias, dinv)
        rows_a = _sc_rows(hh, src_g, dst_s)[0].reshape(NC, NACC, D)
        h1, n = _tc(_h1_body, [(NACC, D), (NACC,)])(sp, dinv, rows_a)
        rows_b, t1p = _sc_rows_scal(h1, n, dst_g, src_s)
        a, ah, s1 = _tc(_coef_body, [(NACC,), (NACC, D), (NACC, D)])(
            h1, n, deg_out, rows_b.reshape(NC, NACC, D), t1p.reshape(NC, NACC))
        rows_c, s3p = _sc_rows_scal(ah, a, src_g, dst_s)
        h = _tc(_grad_body, [(NACC, D)])(
            h1, a, deg_out, s1, rows_c.reshape(NC, NACC, D),
            s3p.reshape(NC, NACC))[0]

    hh, sp = _tc(_mm_body, [(NACC, D), (NACC, D)])(h, W_out, b_out, dinv)
    rows_f = _sc_rows(hh, src_g, dst_s)[0].reshape(NC, NACC, D)
    emb = _tc(_fin_body, [(NACC, D)])(sp, dinv, rows_f)[0]
    return emb[:N_NODES]


# speedup vs baseline: 1.6109x; 1.0142x over previous
import functools

import jax
import jax.numpy as jnp
from jax import lax
from jax.experimental import pallas as pl
from jax.experimental.pallas import tpu as pltpu
from jax.experimental.pallas import tpu_sc as plsc

N_NODES = 10000
D = 128
N_EDGES = 320000
NC = 2
NS = 16
NW = NC * NS
CH = 128
NCHUNK = 80
EPW = NCHUNK * CH
E_PAD = NW * EPW
NACC = 10240
RPT = NACC // NS
TEMP = 10.0
WGT = 1.0

_mesh = plsc.VectorSubcoreMesh(core_axis_name="c", subcore_axis_name="s")
_f32 = jnp.float32


@functools.partial(
    pl.kernel,
    out_type=[jax.ShapeDtypeStruct((NC * NACC, D), _f32)],
    mesh=_mesh,
    scratch_types=[
        pltpu.VMEM((NCHUNK, CH), jnp.int32),
        pltpu.VMEM((2, 1, CH), jnp.int32),
        pltpu.VMEM((2, CH, D), _f32),
        pltpu.VMEM((16, D), _f32),
        pltpu.VMEM_SHARED((NACC, D), _f32),
        pltpu.SemaphoreType.DMA,
        pltpu.SemaphoreType.DMA,
        pltpu.SemaphoreType.DMA,
        pltpu.SemaphoreType.DMA,
        pltpu.SemaphoreType.DMA,
        pltpu.SemaphoreType.DMA,
    ],
)
def _sc_rows(table_h, gidx_h, sidx_h, out_h,
             gidx_v, sidx_v, rows_v, zrow_v, acc_r, sem_g, sem_g1,
             sem_w0, sem_w1, sem_i0, sem_i1):
    cid = lax.axis_index("c")
    sid = lax.axis_index("s")
    wid = sid * NC + cid
    tb = sid * RPT
    for r in range(16):
        for k in range(D // 16):
            zrow_v[r, pl.ds(k * 16, 16)] = jnp.zeros((16,), _f32)

    def zstep(j, carry):
        pltpu.async_copy(zrow_v, acc_r.at[pl.ds(tb + j * 16, 16)], sem_g)
        return carry

    lax.fori_loop(0, RPT // 16, zstep, 0)
    pltpu.sync_copy(gidx_h.at[wid], gidx_v)

    def zdrain(j, carry):
        pltpu.make_async_copy(
            zrow_v, acc_r.at[pl.ds(tb + j * 16, 16)], sem_g).wait()
        return carry

    lax.fori_loop(0, RPT // 16, zdrain, 0)
    plsc.subcore_barrier()
    HC = CH // 2
    gsems = (sem_g, sem_g1)
    wsems = (sem_w0, sem_w1)
    isems = (sem_i0, sem_i1)

    def gissue(ci, buf):
        for h in range(2):
            pltpu.async_copy(table_h.at[gidx_v.at[ci, pl.ds(h * HC, HC)]],
                             rows_v.at[buf, pl.ds(h * HC, HC)], gsems[h])

    def gwait(ci, buf):
        for h in range(2):
            pltpu.make_async_copy(
                table_h.at[gidx_v.at[ci, pl.ds(h * HC, HC)]],
                rows_v.at[buf, pl.ds(h * HC, HC)], gsems[h]).wait()

    gissue(0, 0)
    pltpu.async_copy(sidx_h.at[wid, pl.ds(0, 1)], sidx_v.at[0], sem_i0)
    pltpu.async_copy(sidx_h.at[wid, pl.ds(1, 1)], sidx_v.at[1], sem_i1)

    def step(g, carry):
        for b in range(2):
            ci = 2 * g + b
            gwait(ci, b)
            nx = jnp.minimum(ci + 1, NCHUNK - 1)
            gissue(nx, 1 - b)
            pltpu.make_async_copy(
                sidx_h.at[wid, pl.ds(ci, 1)], sidx_v.at[b], isems[b]).wait()
            for h in range(2):
                pltpu.async_copy(rows_v.at[b, pl.ds(h * HC, HC)],
                                 acc_r.at[sidx_v.at[b, 0, pl.ds(h * HC, HC)]],
                                 wsems[h], add=True)
            for h in range(2):
                pltpu.make_async_copy(
                    rows_v.at[b, pl.ds(h * HC, HC)],
                    acc_r.at[sidx_v.at[b, 0, pl.ds(h * HC, HC)]],
                    wsems[h]).wait()
            nx2 = jnp.minimum(ci + 2, NCHUNK - 1)
            pltpu.async_copy(sidx_h.at[wid, pl.ds(nx2, 1)], sidx_v.at[b],
                             isems[b])
        return carry

    lax.fori_loop(0, NCHUNK // 2, step, 0)
    gwait(NCHUNK - 1, 0)
    pltpu.make_async_copy(
        sidx_h.at[wid, pl.ds(0, 1)], sidx_v.at[0], sem_i0).wait()
    pltpu.make_async_copy(
        sidx_h.at[wid, pl.ds(0, 1)], sidx_v.at[1], sem_i1).wait()
    plsc.subcore_barrier()
    pltpu.sync_copy(acc_r.at[pl.ds(tb, RPT)],
                    out_h.at[pl.ds(cid * NACC + tb, RPT)])


@functools.partial(
    pl.kernel,
    out_type=[jax.ShapeDtypeStruct((NC * NACC, D), _f32),
              jax.ShapeDtypeStruct((NC * NACC,), _f32)],
    mesh=_mesh,
    scratch_types=[
        pltpu.VMEM((NCHUNK, CH), jnp.int32),
        pltpu.VMEM((2, 1, CH), jnp.int32),
        pltpu.VMEM((2, CH, D), _f32),
        pltpu.VMEM((2, CH), _f32),
        pltpu.VMEM((16, D), _f32),
        pltpu.VMEM((RPT,), _f32),
        pltpu.VMEM_SHARED((NACC, D), _f32),
        pltpu.VMEM_SHARED((NACC,), _f32),
        pltpu.SemaphoreType.DMA,
        pltpu.SemaphoreType.DMA,
        pltpu.SemaphoreType.DMA,
        pltpu.SemaphoreType.DMA,
    ],
)
def _sc_rows_scal(table_h, tab1_h, gidx_h, sidx_h, out_h, out1_h,
                  gidx_v, sidx_v, rows_v, scal_v, zrow_v, bnc_v, acc_r, acc_s,
                  sem_r, sem_s, sem_i0, sem_i1):
    cid = lax.axis_index("c")
    sid = lax.axis_index("s")
    wid = sid * NC + cid
    tb = sid * RPT
    for r in range(16):
        for k in range(D // 16):
            zrow_v[r, pl.ds(k * 16, 16)] = jnp.zeros((16,), _f32)

    def zstep(j, carry):
        pltpu.async_copy(zrow_v, acc_r.at[pl.ds(tb + j * 16, 16)], sem_r)
        return carry

    lax.fori_loop(0, RPT // 16, zstep, 0)
    for k in range(RPT // 16):
        bnc_v[pl.ds(k * 16, 16)] = jnp.zeros((16,), _f32)
    pltpu.sync_copy(bnc_v, acc_s.at[pl.ds(tb, RPT)])
    pltpu.sync_copy(gidx_h.at[wid], gidx_v)

    def zdrain(j, carry):
        pltpu.make_async_copy(
            zrow_v, acc_r.at[pl.ds(tb + j * 16, 16)], sem_r).wait()
        return carry

    lax.fori_loop(0, RPT // 16, zdrain, 0)
    plsc.subcore_barrier()
    pltpu.async_copy(table_h.at[gidx_v.at[0]], rows_v.at[0], sem_r)
    pltpu.async_copy(tab1_h.at[gidx_v.at[0]], scal_v.at[0], sem_s)
    pltpu.async_copy(sidx_h.at[wid, pl.ds(0, 1)], sidx_v.at[0], sem_i0)
    pltpu.async_copy(sidx_h.at[wid, pl.ds(1, 1)], sidx_v.at[1], sem_i1)
    isems = (sem_i0, sem_i1)

    def step(g, carry):
        for b in range(2):
            ci = 2 * g + b
            pltpu.make_async_copy(
                table_h.at[gidx_v.at[ci]], rows_v.at[b], sem_r).wait()
            pltpu.make_async_copy(
                tab1_h.at[gidx_v.at[ci]], scal_v.at[b], sem_s).wait()
            nx = jnp.minimum(ci + 1, NCHUNK - 1)
            pltpu.async_copy(table_h.at[gidx_v.at[nx]], rows_v.at[1 - b], sem_r)
            pltpu.async_copy(tab1_h.at[gidx_v.at[nx]], scal_v.at[1 - b], sem_s)
            pltpu.make_async_copy(
                sidx_h.at[wid, pl.ds(ci, 1)], sidx_v.at[b], isems[b]).wait()
            pltpu.sync_copy(rows_v.at[b], acc_r.at[sidx_v.at[b, 0]], add=True)
            pltpu.sync_copy(scal_v.at[b], acc_s.at[sidx_v.at[b, 0]], add=True)
            nx2 = jnp.minimum(ci + 2, NCHUNK - 1)
            pltpu.async_copy(sidx_h.at[wid, pl.ds(nx2, 1)], sidx_v.at[b],
                             isems[b])
        return carry

    lax.fori_loop(0, NCHUNK // 2, step, 0)
    pltpu.make_async_copy(
        table_h.at[gidx_v.at[NCHUNK - 1]], rows_v.at[0], sem_r).wait()
    pltpu.make_async_copy(
        tab1_h.at[gidx_v.at[NCHUNK - 1]], scal_v.at[0], sem_s).wait()
    pltpu.make_async_copy(
        sidx_h.at[wid, pl.ds(0, 1)], sidx_v.at[0], sem_i0).wait()
    pltpu.make_async_copy(
        sidx_h.at[wid, pl.ds(0, 1)], sidx_v.at[1], sem_i1).wait()
    plsc.subcore_barrier()
    pltpu.sync_copy(acc_r.at[pl.ds(tb, RPT)],
                    out_h.at[pl.ds(cid * NACC + tb, RPT)])
    pltpu.sync_copy(acc_s.at[pl.ds(tb, RPT)], bnc_v)
    pltpu.sync_copy(bnc_v, out1_h.at[pl.ds(cid * NACC + tb, RPT)])


@functools.partial(
    pl.kernel,
    out_type=[jax.ShapeDtypeStruct((NC * NACC,), _f32),
              jax.ShapeDtypeStruct((NC * NACC,), _f32)],
    mesh=_mesh,
    scratch_types=[
        pltpu.VMEM((NCHUNK, CH), jnp.int32),
        pltpu.VMEM((NCHUNK, CH), jnp.int32),
        pltpu.VMEM((CH,), _f32),
        pltpu.VMEM((RPT,), _f32),
        pltpu.VMEM_SHARED((NACC,), _f32),
        pltpu.VMEM_SHARED((NACC,), _f32),
        pltpu.SemaphoreType.DMA,
        pltpu.SemaphoreType.DMA,
    ],
)
def _sc_deg(didx_h, oidx_h, outd_h, outo_h,
            didx_v, oidx_v, ones_v, bnc_v, acc_d, acc_o, sem_d, sem_o):
    cid = lax.axis_index("c")
    sid = lax.axis_index("s")
    wid = sid * NC + cid
    tb = sid * RPT
    for k in range(RPT // 16):
        bnc_v[pl.ds(k * 16, 16)] = jnp.zeros((16,), _f32)
    pltpu.sync_copy(bnc_v, acc_d.at[pl.ds(tb, RPT)])
    pltpu.sync_copy(bnc_v, acc_o.at[pl.ds(tb, RPT)])
    pltpu.sync_copy(didx_h.at[wid], didx_v)
    pltpu.sync_copy(oidx_h.at[wid], oidx_v)
    for k in range(CH // 16):
        ones_v[pl.ds(k * 16, 16)] = jnp.full((16,), 1.0, _f32)
    plsc.subcore_barrier()

    def step(ci, carry):
        pltpu.async_copy(ones_v, acc_d.at[didx_v.at[ci]], sem_d, add=True)
        pltpu.async_copy(ones_v, acc_o.at[oidx_v.at[ci]], sem_o, add=True)
        return carry

    lax.fori_loop(0, NCHUNK, step, 0)

    def drain(ci, carry):
        pltpu.make_async_copy(ones_v, acc_d.at[didx_v.at[ci]], sem_d).wait()
        pltpu.make_async_copy(ones_v, acc_o.at[oidx_v.at[ci]], sem_o).wait()
        return carry

    lax.fori_loop(0, NCHUNK, drain, 0)
    plsc.subcore_barrier()
    pltpu.sync_copy(acc_d.at[pl.ds(tb, RPT)], bnc_v)
    pltpu.sync_copy(bnc_v, outd_h.at[pl.ds(cid * NACC + tb, RPT)])
    pltpu.sync_copy(acc_o.at[pl.ds(tb, RPT)], bnc_v)
    pltpu.sync_copy(bnc_v, outo_h.at[pl.ds(cid * NACC + tb, RPT)])


def _tc(body, n_out):
    shapes = []
    for s in n_out:
        shapes.append(jax.ShapeDtypeStruct(s, _f32))
    return pl.pallas_call(body, out_shape=shapes)


def _deg_body(dp_ref, op_ref, dinv_o, dout_o):
    deg = dp_ref[0] + dp_ref[1] + 1.0
    dinv_o[...] = lax.rsqrt(jnp.maximum(deg, 1e-12))
    dout_o[...] = op_ref[0] + op_ref[1]


def _mm_body(x_ref, w_ref, b_ref, dinv_ref, hh_o, sp_o):
    h = jnp.dot(x_ref[...], w_ref[...], preferred_element_type=_f32)
    dv = dinv_ref[...][:, None]
    hh_o[...] = dv * h
    sp_o[...] = b_ref[...][None, :] + dv * dv * h


def _h1_body(sp_ref, dinv_ref, rp_ref, h1_o, n_o):
    h1 = sp_ref[...] + dinv_ref[...][:, None] * (rp_ref[0] + rp_ref[1])
    h1_o[...] = h1
    n_o[...] = jnp.sum(h1 * h1, axis=1)


def _fin_body(sp_ref, dinv_ref, rp_ref, o_ref):
    o_ref[...] = sp_ref[...] + dinv_ref[...][:, None] * (rp_ref[0] + rp_ref[1])


def _coef_body(h1_ref, n_ref, dout_ref, s1p_ref, t1p_ref, a_o, ah_o, s1_o):
    h1 = h1_ref[...]
    s1 = s1p_ref[0] + s1p_ref[1]
    s1_o[...] = s1
    t1 = t1p_ref[0] + t1p_ref[1]
    e = 0.5 * dout_ref[...] * n_ref[...] - jnp.sum(h1 * s1, axis=1) + 0.5 * t1
    valid = lax.broadcasted_iota(jnp.int32, (NACC,), 0) < N_NODES
    m = jnp.max(jnp.where(valid, e, -jnp.inf))
    c = jnp.maximum(m, 1e-9)
    et = e / c
    z = jnp.where(valid, -et / TEMP, -jnp.inf)
    p = jnp.exp(z - jnp.max(z))
    p = p / jnp.sum(p)
    g = -(jnp.log(p + 1e-12) + p / (p + 1e-12))
    dz = p * (g - jnp.sum(g * p))
    q = -dz / TEMP
    mask = jnp.where(valid & (e == m), 1.0, 0.0)
    nt = jnp.maximum(jnp.sum(mask), 1.0)
    dcdm = jnp.where(m > 1e-9, 1.0, jnp.where(m == 1e-9, 0.5, 0.0))
    beta = dcdm * (-jnp.sum(q * et) / c)
    a = jnp.where(valid, q / c + beta * mask / nt, 0.0)
    a_o[...] = a
    ah_o[...] = a[:, None] * h1


def _grad_body(h1_ref, a_ref, dout_ref, s1_ref, s2p_ref, s3p_ref, o_ref):
    h1 = h1_ref[...]
    s2 = s2p_ref[0] + s2p_ref[1]
    s3 = (s3p_ref[0] + s3p_ref[1])[:, None]
    av = a_ref[...][:, None]
    grd = av * (dout_ref[...][:, None] * h1 - s1_ref[...]) - s2 + s3 * h1
    o_ref[...] = jnp.maximum(h1 + WGT * grd, 0.0)


def kernel(x, edge_index, W1, b1, W2, b2, W_out, b_out):
    src = edge_index[0]
    dst = edge_index[1]
    npad = E_PAD - N_EDGES
    ar = jnp.arange(npad, dtype=jnp.int32)
    pad_g = ar % N_NODES
    pad_s = N_NODES + ar % (NACC - N_NODES)
    src_g = jnp.concatenate([src, pad_g]).reshape(NW, NCHUNK, CH)
    src_s = jnp.concatenate([src, pad_s]).reshape(NW, NCHUNK, CH)
    dst_g = jnp.concatenate([dst, pad_g]).reshape(NW, NCHUNK, CH)
    dst_s = jnp.concatenate([dst, pad_s]).reshape(NW, NCHUNK, CH)
    xp = jnp.pad(x, ((0, NACC - N_NODES), (0, 0)))

    degp, degop = _sc_deg(dst_s, src_s)
    dinv, deg_out = _tc(_deg_body, [(NACC,), (NACC,)])(
        degp.reshape(NC, NACC), degop.reshape(NC, NACC))

    h = xp
    for wgt, bias in ((W1, b1), (W2, b2)):
        hh, sp = _tc(_mm_body, [(NACC, D), (NACC, D)])(h, wgt, bias, dinv)
        rows_a = _sc_rows(hh, src_g, dst_s)[0].reshape(NC, NACC, D)
        h1, n = _tc(_h1_body, [(NACC, D), (NACC,)])(sp, dinv, rows_a)
        rows_b, t1p = _sc_rows_scal(h1, n, dst_g, src_s)
        a, ah, s1 = _tc(_coef_body, [(NACC,), (NACC, D), (NACC, D)])(
            h1, n, deg_out, rows_b.reshape(NC, NACC, D), t1p.reshape(NC, NACC))
        rows_c, s3p = _sc_rows_scal(ah, a, src_g, dst_s)
        h = _tc(_grad_body, [(NACC, D)])(
            h1, a, deg_out, s1, rows_c.reshape(NC, NACC, D),
            s3p.reshape(NC, NACC))[0]

    hh, sp = _tc(_mm_body, [(NACC, D), (NACC, D)])(h, W_out, b_out, dinv)
    rows_f = _sc_rows(hh, src_g, dst_s)[0].reshape(NC, NACC, D)
    emb = _tc(_fin_body, [(NACC, D)])(sp, dinv, rows_f)[0]
    return emb[:N_NODES]


# speedup vs baseline: 1.6118x; 1.0005x over previous
import functools

import jax
import jax.numpy as jnp
from jax import lax
from jax.experimental import pallas as pl
from jax.experimental.pallas import tpu as pltpu
from jax.experimental.pallas import tpu_sc as plsc

N_NODES = 10000
D = 128
N_EDGES = 320000
NC = 2
NS = 16
NW = NC * NS
CH = 128
NCHUNK = 80
EPW = NCHUNK * CH
E_PAD = NW * EPW
NACC = 10240
RPT = NACC // NS
TEMP = 10.0
WGT = 1.0

_mesh = plsc.VectorSubcoreMesh(core_axis_name="c", subcore_axis_name="s")
_f32 = jnp.float32


@functools.partial(
    pl.kernel,
    out_type=[jax.ShapeDtypeStruct((NC * NACC, D), _f32)],
    mesh=_mesh,
    scratch_types=[
        pltpu.VMEM((NCHUNK, CH), jnp.int32),
        pltpu.VMEM((2, 1, CH), jnp.int32),
        pltpu.VMEM((2, CH, D), _f32),
        pltpu.VMEM((16, D), _f32),
        pltpu.VMEM_SHARED((NACC, D), _f32),
        pltpu.SemaphoreType.DMA,
        pltpu.SemaphoreType.DMA,
        pltpu.SemaphoreType.DMA,
        pltpu.SemaphoreType.DMA,
        pltpu.SemaphoreType.DMA,
        pltpu.SemaphoreType.DMA,
    ],
)
def _sc_rows(table_h, gidx_h, sidx_h, out_h,
             gidx_v, sidx_v, rows_v, zrow_v, acc_r, sem_g, sem_g1,
             sem_w0, sem_w1, sem_i0, sem_i1):
    cid = lax.axis_index("c")
    sid = lax.axis_index("s")
    wid = sid * NC + cid
    tb = sid * RPT
    for r in range(16):
        for k in range(D // 16):
            zrow_v[r, pl.ds(k * 16, 16)] = jnp.zeros((16,), _f32)

    def zstep(j, carry):
        pltpu.async_copy(zrow_v, acc_r.at[pl.ds(tb + j * 16, 16)], sem_g)
        return carry

    lax.fori_loop(0, RPT // 16, zstep, 0)
    pltpu.sync_copy(gidx_h.at[wid], gidx_v)

    def zdrain(j, carry):
        pltpu.make_async_copy(
            zrow_v, acc_r.at[pl.ds(tb + j * 16, 16)], sem_g).wait()
        return carry

    lax.fori_loop(0, RPT // 16, zdrain, 0)
    plsc.subcore_barrier()
    HC = CH // 2
    gsems = (sem_g, sem_g1)
    wsems = (sem_w0, sem_w1)
    isems = (sem_i0, sem_i1)

    def gissue(ci, buf):
        for h in range(2):
            pltpu.async_copy(table_h.at[gidx_v.at[ci, pl.ds(h * HC, HC)]],
                             rows_v.at[buf, pl.ds(h * HC, HC)], gsems[h])

    def gwait(ci, buf):
        for h in range(2):
            pltpu.make_async_copy(
                table_h.at[gidx_v.at[ci, pl.ds(h * HC, HC)]],
                rows_v.at[buf, pl.ds(h * HC, HC)], gsems[h]).wait()

    gissue(0, 0)
    pltpu.async_copy(sidx_h.at[wid, pl.ds(0, 1)], sidx_v.at[0], sem_i0)
    pltpu.async_copy(sidx_h.at[wid, pl.ds(1, 1)], sidx_v.at[1], sem_i1)

    def step(g, carry):
        for b in range(2):
            ci = 2 * g + b
            gwait(ci, b)
            nx = jnp.minimum(ci + 1, NCHUNK - 1)
            gissue(nx, 1 - b)
            pltpu.make_async_copy(
                sidx_h.at[wid, pl.ds(ci, 1)], sidx_v.at[b], isems[b]).wait()
            for h in range(2):
                pltpu.async_copy(rows_v.at[b, pl.ds(h * HC, HC)],
                                 acc_r.at[sidx_v.at[b, 0, pl.ds(h * HC, HC)]],
                                 wsems[h], add=True)
            for h in range(2):
                pltpu.make_async_copy(
                    rows_v.at[b, pl.ds(h * HC, HC)],
                    acc_r.at[sidx_v.at[b, 0, pl.ds(h * HC, HC)]],
                    wsems[h]).wait()
            nx2 = jnp.minimum(ci + 2, NCHUNK - 1)
            pltpu.async_copy(sidx_h.at[wid, pl.ds(nx2, 1)], sidx_v.at[b],
                             isems[b])
        return carry

    lax.fori_loop(0, NCHUNK // 2, step, 0)
    gwait(NCHUNK - 1, 0)
    pltpu.make_async_copy(
        sidx_h.at[wid, pl.ds(0, 1)], sidx_v.at[0], sem_i0).wait()
    pltpu.make_async_copy(
        sidx_h.at[wid, pl.ds(0, 1)], sidx_v.at[1], sem_i1).wait()
    plsc.subcore_barrier()
    pltpu.sync_copy(acc_r.at[pl.ds(tb, RPT)],
                    out_h.at[pl.ds(cid * NACC + tb, RPT)])


@functools.partial(
    pl.kernel,
    out_type=[jax.ShapeDtypeStruct((NC * NACC, D), _f32),
              jax.ShapeDtypeStruct((NC * NACC,), _f32)],
    mesh=_mesh,
    scratch_types=[
        pltpu.VMEM((NCHUNK, CH), jnp.int32),
        pltpu.VMEM((2, 1, CH), jnp.int32),
        pltpu.VMEM((2, CH, D), _f32),
        pltpu.VMEM((2, CH), _f32),
        pltpu.VMEM((16, D), _f32),
        pltpu.VMEM((RPT,), _f32),
        pltpu.VMEM_SHARED((NACC, D), _f32),
        pltpu.VMEM_SHARED((NACC,), _f32),
        pltpu.SemaphoreType.DMA,
        pltpu.SemaphoreType.DMA,
        pltpu.SemaphoreType.DMA,
        pltpu.SemaphoreType.DMA,
        pltpu.SemaphoreType.DMA,
        pltpu.SemaphoreType.DMA,
        pltpu.SemaphoreType.DMA,
    ],
)
def _sc_rows_scal(table_h, tab1_h, gidx_h, sidx_h, out_h, out1_h,
                  gidx_v, sidx_v, rows_v, scal_v, zrow_v, bnc_v, acc_r, acc_s,
                  sem_r, sem_r1, sem_w0, sem_w1, sem_s, sem_i0, sem_i1):
    cid = lax.axis_index("c")
    sid = lax.axis_index("s")
    wid = sid * NC + cid
    tb = sid * RPT
    for r in range(16):
        for k in range(D // 16):
            zrow_v[r, pl.ds(k * 16, 16)] = jnp.zeros((16,), _f32)

    def zstep(j, carry):
        pltpu.async_copy(zrow_v, acc_r.at[pl.ds(tb + j * 16, 16)], sem_r)
        return carry

    lax.fori_loop(0, RPT // 16, zstep, 0)
    for k in range(RPT // 16):
        bnc_v[pl.ds(k * 16, 16)] = jnp.zeros((16,), _f32)
    pltpu.sync_copy(bnc_v, acc_s.at[pl.ds(tb, RPT)])
    pltpu.sync_copy(gidx_h.at[wid], gidx_v)

    def zdrain(j, carry):
        pltpu.make_async_copy(
            zrow_v, acc_r.at[pl.ds(tb + j * 16, 16)], sem_r).wait()
        return carry

    lax.fori_loop(0, RPT // 16, zdrain, 0)
    plsc.subcore_barrier()
    HC = CH // 2
    gsems = (sem_r, sem_r1)
    wsems = (sem_w0, sem_w1)
    isems = (sem_i0, sem_i1)

    def gissue(ci, buf):
        for h in range(2):
            pltpu.async_copy(table_h.at[gidx_v.at[ci, pl.ds(h * HC, HC)]],
                             rows_v.at[buf, pl.ds(h * HC, HC)], gsems[h])

    def gwait(ci, buf):
        for h in range(2):
            pltpu.make_async_copy(
                table_h.at[gidx_v.at[ci, pl.ds(h * HC, HC)]],
                rows_v.at[buf, pl.ds(h * HC, HC)], gsems[h]).wait()

    gissue(0, 0)
    pltpu.async_copy(tab1_h.at[gidx_v.at[0]], scal_v.at[0], sem_s)
    pltpu.async_copy(sidx_h.at[wid, pl.ds(0, 1)], sidx_v.at[0], sem_i0)
    pltpu.async_copy(sidx_h.at[wid, pl.ds(1, 1)], sidx_v.at[1], sem_i1)

    def step(g, carry):
        for b in range(2):
            ci = 2 * g + b
            gwait(ci, b)
            pltpu.make_async_copy(
                tab1_h.at[gidx_v.at[ci]], scal_v.at[b], sem_s).wait()
            nx = jnp.minimum(ci + 1, NCHUNK - 1)
            gissue(nx, 1 - b)
            pltpu.async_copy(tab1_h.at[gidx_v.at[nx]], scal_v.at[1 - b], sem_s)
            pltpu.make_async_copy(
                sidx_h.at[wid, pl.ds(ci, 1)], sidx_v.at[b], isems[b]).wait()
            for h in range(2):
                pltpu.async_copy(rows_v.at[b, pl.ds(h * HC, HC)],
                                 acc_r.at[sidx_v.at[b, 0, pl.ds(h * HC, HC)]],
                                 wsems[h], add=True)
            pltpu.sync_copy(scal_v.at[b], acc_s.at[sidx_v.at[b, 0]], add=True)
            for h in range(2):
                pltpu.make_async_copy(
                    rows_v.at[b, pl.ds(h * HC, HC)],
                    acc_r.at[sidx_v.at[b, 0, pl.ds(h * HC, HC)]],
                    wsems[h]).wait()
            nx2 = jnp.minimum(ci + 2, NCHUNK - 1)
            pltpu.async_copy(sidx_h.at[wid, pl.ds(nx2, 1)], sidx_v.at[b],
                             isems[b])
        return carry

    lax.fori_loop(0, NCHUNK // 2, step, 0)
    gwait(NCHUNK - 1, 0)
    pltpu.make_async_copy(
        tab1_h.at[gidx_v.at[NCHUNK - 1]], scal_v.at[0], sem_s).wait()
    pltpu.make_async_copy(
        sidx_h.at[wid, pl.ds(0, 1)], sidx_v.at[0], sem_i0).wait()
    pltpu.make_async_copy(
        sidx_h.at[wid, pl.ds(0, 1)], sidx_v.at[1], sem_i1).wait()
    plsc.subcore_barrier()
    pltpu.sync_copy(acc_r.at[pl.ds(tb, RPT)],
                    out_h.at[pl.ds(cid * NACC + tb, RPT)])
    pltpu.sync_copy(acc_s.at[pl.ds(tb, RPT)], bnc_v)
    pltpu.sync_copy(bnc_v, out1_h.at[pl.ds(cid * NACC + tb, RPT)])


@functools.partial(
    pl.kernel,
    out_type=[jax.ShapeDtypeStruct((NC * NACC,), _f32),
              jax.ShapeDtypeStruct((NC * NACC,), _f32)],
    mesh=_mesh,
    scratch_types=[
        pltpu.VMEM((NCHUNK, CH), jnp.int32),
        pltpu.VMEM((NCHUNK, CH), jnp.int32),
        pltpu.VMEM((CH,), _f32),
        pltpu.VMEM((RPT,), _f32),
        pltpu.VMEM_SHARED((NACC,), _f32),
        pltpu.VMEM_SHARED((NACC,), _f32),
        pltpu.SemaphoreType.DMA,
        pltpu.SemaphoreType.DMA,
    ],
)
def _sc_deg(didx_h, oidx_h, outd_h, outo_h,
            didx_v, oidx_v, ones_v, bnc_v, acc_d, acc_o, sem_d, sem_o):
    cid = lax.axis_index("c")
    sid = lax.axis_index("s")
    wid = sid * NC + cid
    tb = sid * RPT
    for k in range(RPT // 16):
        bnc_v[pl.ds(k * 16, 16)] = jnp.zeros((16,), _f32)
    pltpu.sync_copy(bnc_v, acc_d.at[pl.ds(tb, RPT)])
    pltpu.sync_copy(bnc_v, acc_o.at[pl.ds(tb, RPT)])
    pltpu.sync_copy(didx_h.at[wid], didx_v)
    pltpu.sync_copy(oidx_h.at[wid], oidx_v)
    for k in range(CH // 16):
        ones_v[pl.ds(k * 16, 16)] = jnp.full((16,), 1.0, _f32)
    plsc.subcore_barrier()

    def step(ci, carry):
        pltpu.async_copy(ones_v, acc_d.at[didx_v.at[ci]], sem_d, add=True)
        pltpu.async_copy(ones_v, acc_o.at[oidx_v.at[ci]], sem_o, add=True)
        return carry

    lax.fori_loop(0, NCHUNK, step, 0)

    def drain(ci, carry):
        pltpu.make_async_copy(ones_v, acc_d.at[didx_v.at[ci]], sem_d).wait()
        pltpu.make_async_copy(ones_v, acc_o.at[oidx_v.at[ci]], sem_o).wait()
        return carry

    lax.fori_loop(0, NCHUNK, drain, 0)
    plsc.subcore_barrier()
    pltpu.sync_copy(acc_d.at[pl.ds(tb, RPT)], bnc_v)
    pltpu.sync_copy(bnc_v, outd_h.at[pl.ds(cid * NACC + tb, RPT)])
    pltpu.sync_copy(acc_o.at[pl.ds(tb, RPT)], bnc_v)
    pltpu.sync_copy(bnc_v, outo_h.at[pl.ds(cid * NACC + tb, RPT)])


def _tc(body, n_out):
    shapes = []
    for s in n_out:
        shapes.append(jax.ShapeDtypeStruct(s, _f32))
    return pl.pallas_call(body, out_shape=shapes)


def _deg_body(dp_ref, op_ref, dinv_o, dout_o):
    deg = dp_ref[0] + dp_ref[1] + 1.0
    dinv_o[...] = lax.rsqrt(jnp.maximum(deg, 1e-12))
    dout_o[...] = op_ref[0] + op_ref[1]


def _mm_body(x_ref, w_ref, b_ref, dinv_ref, hh_o, sp_o):
    h = jnp.dot(x_ref[...], w_ref[...], preferred_element_type=_f32)
    dv = dinv_ref[...][:, None]
    hh_o[...] = dv * h
    sp_o[...] = b_ref[...][None, :] + dv * dv * h


def _h1_body(sp_ref, dinv_ref, rp_ref, h1_o, n_o):
    h1 = sp_ref[...] + dinv_ref[...][:, None] * (rp_ref[0] + rp_ref[1])
    h1_o[...] = h1
    n_o[...] = jnp.sum(h1 * h1, axis=1)


def _fin_body(sp_ref, dinv_ref, rp_ref, o_ref):
    o_ref[...] = sp_ref[...] + dinv_ref[...][:, None] * (rp_ref[0] + rp_ref[1])


def _coef_body(h1_ref, n_ref, dout_ref, s1p_ref, t1p_ref, a_o, ah_o, s1_o):
    h1 = h1_ref[...]
    s1 = s1p_ref[0] + s1p_ref[1]
    s1_o[...] = s1
    t1 = t1p_ref[0] + t1p_ref[1]
    e = 0.5 * dout_ref[...] * n_ref[...] - jnp.sum(h1 * s1, axis=1) + 0.5 * t1
    valid = lax.broadcasted_iota(jnp.int32, (NACC,), 0) < N_NODES
    m = jnp.max(jnp.where(valid, e, -jnp.inf))
    c = jnp.maximum(m, 1e-9)
    et = e / c
    z = jnp.where(valid, -et / TEMP, -jnp.inf)
    p = jnp.exp(z - jnp.max(z))
    p = p / jnp.sum(p)
    g = -(jnp.log(p + 1e-12) + p / (p + 1e-12))
    dz = p * (g - jnp.sum(g * p))
    q = -dz / TEMP
    mask = jnp.where(valid & (e == m), 1.0, 0.0)
    nt = jnp.maximum(jnp.sum(mask), 1.0)
    dcdm = jnp.where(m > 1e-9, 1.0, jnp.where(m == 1e-9, 0.5, 0.0))
    beta = dcdm * (-jnp.sum(q * et) / c)
    a = jnp.where(valid, q / c + beta * mask / nt, 0.0)
    a_o[...] = a
    ah_o[...] = a[:, None] * h1


def _grad_body(h1_ref, a_ref, dout_ref, s1_ref, s2p_ref, s3p_ref, o_ref):
    h1 = h1_ref[...]
    s2 = s2p_ref[0] + s2p_ref[1]
    s3 = (s3p_ref[0] + s3p_ref[1])[:, None]
    av = a_ref[...][:, None]
    grd = av * (dout_ref[...][:, None] * h1 - s1_ref[...]) - s2 + s3 * h1
    o_ref[...] = jnp.maximum(h1 + WGT * grd, 0.0)


def kernel(x, edge_index, W1, b1, W2, b2, W_out, b_out):
    src = edge_index[0]
    dst = edge_index[1]
    npad = E_PAD - N_EDGES
    ar = jnp.arange(npad, dtype=jnp.int32)
    pad_g = ar % N_NODES
    pad_s = N_NODES + ar % (NACC - N_NODES)
    src_g = jnp.concatenate([src, pad_g]).reshape(NW, NCHUNK, CH)
    src_s = jnp.concatenate([src, pad_s]).reshape(NW, NCHUNK, CH)
    dst_g = jnp.concatenate([dst, pad_g]).reshape(NW, NCHUNK, CH)
    dst_s = jnp.concatenate([dst, pad_s]).reshape(NW, NCHUNK, CH)
    xp = jnp.pad(x, ((0, NACC - N_NODES), (0, 0)))

    degp, degop = _sc_deg(dst_s, src_s)
    dinv, deg_out = _tc(_deg_body, [(NACC,), (NACC,)])(
        degp.reshape(NC, NACC), degop.reshape(NC, NACC))

    h = xp
    for wgt, bias in ((W1, b1), (W2, b2)):
        hh, sp = _tc(_mm_body, [(NACC, D), (NACC, D)])(h, wgt, bias, dinv)
        rows_a = _sc_rows(hh, src_g, dst_s)[0].reshape(NC, NACC, D)
        h1, n = _tc(_h1_body, [(NACC, D), (NACC,)])(sp, dinv, rows_a)
        rows_b, t1p = _sc_rows_scal(h1, n, dst_g, src_s)
        a, ah, s1 = _tc(_coef_body, [(NACC,), (NACC, D), (NACC, D)])(
            h1, n, deg_out, rows_b.reshape(NC, NACC, D), t1p.reshape(NC, NACC))
        rows_c, s3p = _sc_rows_scal(ah, a, src_g, dst_s)
        h = _tc(_grad_body, [(NACC, D)])(
            h1, a, deg_out, s1, rows_c.reshape(NC, NACC, D),
            s3p.reshape(NC, NACC))[0]

    hh, sp = _tc(_mm_body, [(NACC, D), (NACC, D)])(h, W_out, b_out, dinv)
    rows_f = _sc_rows(hh, src_g, dst_s)[0].reshape(NC, NACC, D)
    emb = _tc(_fin_body, [(NACC, D)])(sp, dinv, rows_f)[0]
    return emb[:N_NODES]
